# Initial kernel scaffold; baseline (speedup 1.0000x reference)
import jax
import jax.numpy as jnp
from jax.experimental import pallas as pl


def kernel(C, L, chain_idxs, params):
    raise NotImplementedError("write your pallas kernel here")


# speedup vs baseline: 3.6608x; 3.6608x over previous
import functools

import numpy as np
import jax
import jax.numpy as jnp
from jax import lax
from jax.experimental import pallas as pl
from jax.experimental.pallas import tpu as pltpu
from jax.experimental.pallas import tpu_sc as plsc

NN = 10000
KK = 30
DD = 128
NRBF = 16
NAA = 20
NWL = 8
MINWL, MAXWL = 3.5, 12.0
MINRBF, MAXRBF = 2.0, 22.0

_RB = 200
_NPAN = 10
_PAN = 1024
_NCOL = _NPAN * _PAN

_NBE = 80
_EB = _NBE * KK
_EGRID = NN // _NBE

_NBV = 1000
_VGRID = NN // _NBV

_P = 307200
_NW = 32
_PERW = _P // _NW


def _gelu(x):
    return jax.nn.gelu(x)


def _mm(a, b):
    return lax.dot_general(a, b, (((1,), (0,)), ((), ())),
                           precision=lax.Precision.HIGHEST,
                           preferred_element_type=jnp.float32)


def _ln(x, g, b):
    mu = jnp.mean(x, axis=-1, keepdims=True)
    var = jnp.mean((x - mu) ** 2, axis=-1, keepdims=True)
    return (x - mu) * lax.rsqrt(var + 1e-5) * g + b


def _knn_body(car_ref, cat_ref, idx_ref, vals_ref):
    car = car_ref[...]
    x2r = ((car[:, 0:1] * car[:, 0:1] + car[:, 2:3] * car[:, 2:3])
           + car[:, 1:2] * car[:, 1:2])
    carb = car.astype(jnp.bfloat16).astype(jnp.float32)

    def init(p, _):
        catp = cat_ref[p]
        x2c = ((catp[0:1, :] * catp[0:1, :] + catp[2:3, :] * catp[2:3, :])
               + catp[1:2, :] * catp[1:2, :])
        catb = catp.astype(jnp.bfloat16).astype(jnp.float32)
        p0 = carb[:, 0:1] * catb[0:1, :]
        p1 = carb[:, 1:2] * catb[1:2, :]
        p2 = carb[:, 2:3] * catb[2:3, :]
        s1 = p0 + p1
        bp = s1 - p0
        e1 = (p0 - (s1 - bp)) + (p1 - bp)
        s2 = s1 + p2
        bp2 = s2 - s1
        e2 = (s1 - (s2 - bp2)) + (p2 - bp2)
        dot = s2 + (e1 + e2)
        col = lax.broadcasted_iota(jnp.int32, (_RB, _PAN), 1) + p * _PAN
        d2 = x2r + x2c - 2.0 * dot
        vals_ref[p] = jnp.where(col >= NN, 1e30, d2)
        return _

    lax.fori_loop(0, _NPAN, init, 0)

    srow = lax.broadcasted_iota(jnp.int32, (1, KK, _RB), 1)

    def extract(j, _):
        def pmin(p, m):
            return jnp.minimum(m, jnp.min(vals_ref[p], axis=1, keepdims=True))
        m = lax.fori_loop(0, _NPAN, pmin, jnp.full((_RB, 1), 1e30, jnp.float32))

        def pargmin(p, best):
            v = vals_ref[p]
            col = lax.broadcasted_iota(jnp.int32, (_RB, _PAN), 1) + p * _PAN
            cand = jnp.where(v == m, col, jnp.int32(2 ** 30))
            return jnp.minimum(best, jnp.min(cand, axis=1))
        sel = lax.fori_loop(0, _NPAN, pargmin,
                            jnp.full((_RB,), 2 ** 30, jnp.int32))

        idx_ref[...] = jnp.where(srow == j, sel[None, None, :], idx_ref[...])

        def pupd(p, _c):
            v = vals_ref[p]
            col = lax.broadcasted_iota(jnp.int32, (_RB, _PAN), 1) + p * _PAN
            vals_ref[p] = jnp.where(col == sel[:, None], 1e30, v)
            return _c
        lax.fori_loop(0, _NPAN, pupd, 0)
        return _

    lax.fori_loop(0, KK, extract, 0)


def _knn(car, cat3):
    return pl.pallas_call(
        _knn_body,
        grid=(NN // _RB,),
        in_specs=[
            pl.BlockSpec((_RB, 8), lambda i: (i, 0)),
            pl.BlockSpec((_NPAN, 8, _PAN), lambda i: (0, 0, 0)),
        ],
        out_specs=pl.BlockSpec((1, KK, _RB), lambda i: (i, 0, 0)),
        out_shape=jax.ShapeDtypeStruct((NN // _RB, KK, _RB), jnp.int32),
        scratch_shapes=[pltpu.VMEM((_NPAN, _RB, _PAN), jnp.float32)],
    )(car, cat3)


@functools.lru_cache(maxsize=None)
def _sc_gather(dt, chunk):
    mesh = plsc.VectorSubcoreMesh(core_axis_name="c", subcore_axis_name="s")
    nit = _PERW // chunk

    @functools.partial(
        pl.kernel, mesh=mesh,
        out_type=jax.ShapeDtypeStruct((_P, dt), jnp.float32),
        scratch_types=[
            pltpu.VMEM((chunk,), jnp.int32),
            pltpu.VMEM((chunk, dt), jnp.float32),
            pltpu.SemaphoreType.DMA,
        ],
    )
    def gk(table_hbm, idx_hbm, out_hbm, idx_v, rows_v, sem):
        wid = lax.axis_index("s") * 2 + lax.axis_index("c")
        base = wid * _PERW

        def body(t, carry):
            off = base + t * chunk
            pltpu.sync_copy(idx_hbm.at[pl.ds(off, chunk)], idx_v)
            pltpu.async_copy(table_hbm.at[idx_v], rows_v, sem).wait()
            pltpu.sync_copy(rows_v, out_hbm.at[pl.ds(off, chunk)])
            return carry

        lax.fori_loop(0, nit, body, 0)

    return gk


def _gather_rows(table, idx_pad, chunk):
    return _sc_gather(table.shape[1], chunk)(table, idx_pad)


def _msg_body(e_ref, gg_ref, a_ref, r_ref, rt_ref,
              we_ref, be_ref, w2_ref, b2_ref, out_ref):
    pre = (_mm(r_ref[...], a_ref[...]) + gg_ref[...]
           + _mm(e_ref[...], we_ref[...]) + be_ref[...])
    m = _gelu(pre)
    y = _gelu(_mm(m, w2_ref[...]) + b2_ref[...])
    out_ref[...] = _mm(rt_ref[...], y) * (1.0 / KK)


def _msg_kernel(E, Gg, A, R, Rt, we, be, w2, b2):
    return pl.pallas_call(
        _msg_body,
        grid=(_EGRID,),
        in_specs=[
            pl.BlockSpec((_EB, DD), lambda i: (i, 0)),
            pl.BlockSpec((_EB, DD), lambda i: (i, 0)),
            pl.BlockSpec((_NBE, DD), lambda i: (i, 0)),
            pl.BlockSpec((_EB, _NBE), lambda i: (0, 0)),
            pl.BlockSpec((_NBE, _EB), lambda i: (0, 0)),
            pl.BlockSpec((DD, DD), lambda i: (0, 0)),
            pl.BlockSpec((1, DD), lambda i: (0, 0)),
            pl.BlockSpec((DD, DD), lambda i: (0, 0)),
            pl.BlockSpec((1, DD), lambda i: (0, 0)),
        ],
        out_specs=pl.BlockSpec((_NBE, DD), lambda i: (i, 0)),
        out_shape=jax.ShapeDtypeStruct((NN, DD), jnp.float32),
    )(E, Gg, A, R, Rt, we, be, w2, b2)


def _edge_upd_body(e_ref, gg_ref, a_ref, r_ref,
                   we_ref, be_ref, w2_ref, b2_ref, g_ref, gb_ref, out_ref):
    pre = (_mm(r_ref[...], a_ref[...]) + gg_ref[...]
           + _mm(e_ref[...], we_ref[...]) + be_ref[...])
    h = _mm(_gelu(pre), w2_ref[...]) + b2_ref[...]
    out_ref[...] = _ln(e_ref[...] + h, g_ref[...], gb_ref[...])


def _edge_upd_kernel(E, Gg, A, R, we, be, w2, b2, g, gb):
    return pl.pallas_call(
        _edge_upd_body,
        grid=(_EGRID,),
        in_specs=[
            pl.BlockSpec((_EB, DD), lambda i: (i, 0)),
            pl.BlockSpec((_EB, DD), lambda i: (i, 0)),
            pl.BlockSpec((_NBE, DD), lambda i: (i, 0)),
            pl.BlockSpec((_EB, _NBE), lambda i: (0, 0)),
            pl.BlockSpec((DD, DD), lambda i: (0, 0)),
            pl.BlockSpec((1, DD), lambda i: (0, 0)),
            pl.BlockSpec((DD, DD), lambda i: (0, 0)),
            pl.BlockSpec((1, DD), lambda i: (0, 0)),
            pl.BlockSpec((1, DD), lambda i: (0, 0)),
            pl.BlockSpec((1, DD), lambda i: (0, 0)),
        ],
        out_specs=pl.BlockSpec((_EB, DD), lambda i: (i, 0)),
        out_shape=jax.ShapeDtypeStruct((NN * KK, DD), jnp.float32),
    )(E, Gg, A, R, we, be, w2, b2, g, gb)


def _node_upd_body(nouts, v_ref, s_ref, w3_ref, b3_ref, g1_ref, gb1_ref,
                   f1_ref, fb1_ref, f2_ref, fb2_ref, g2_ref, gb2_ref,
                   *rest):
    u = _ln(v_ref[...] + _mm(s_ref[...], w3_ref[...]) + b3_ref[...],
            g1_ref[...], gb1_ref[...])
    h = _mm(_gelu(_mm(u, f1_ref[...]) + fb1_ref[...]), f2_ref[...]) + fb2_ref[...]
    v2 = _ln(u + h, g2_ref[...], gb2_ref[...])
    wrefs = rest[:-nouts]
    orefs = rest[-nouts:]
    if nouts == 1:
        ow, ob = wrefs
        orefs[0][...] = _mm(v2, ow[...]) + ob[...]
    else:
        orefs[0][...] = v2
        for t in range(nouts - 1):
            w, b = wrefs[2 * t], wrefs[2 * t + 1]
            orefs[t + 1][...] = _mm(v2, w[...]) + b[...]


def _node_upd_kernel(V, S, upd_params, extra_ws, out_dims):
    proj_only = len(out_dims) == 1 and out_dims[0] != DD
    nouts = len(out_dims)
    win = list(upd_params)
    for w, b in extra_ws:
        win += [w, b]
    wspecs = []
    for w in win:
        wspecs.append(pl.BlockSpec(w.shape, lambda i: (0,) * w.ndim))
    out_specs = [pl.BlockSpec((_NBV, d), lambda i: (i, 0)) for d in out_dims]
    out_shape = [jax.ShapeDtypeStruct((NN, d), jnp.float32) for d in out_dims]
    return pl.pallas_call(
        functools.partial(_node_upd_body, nouts),
        grid=(_VGRID,),
        in_specs=[
            pl.BlockSpec((_NBV, DD), lambda i: (i, 0)),
            pl.BlockSpec((_NBV, DD), lambda i: (i, 0)),
        ] + wspecs,
        out_specs=out_specs,
        out_shape=out_shape,
    )(V, S, *win)


def _feat_body(ang_ref, dv_ref, l_ref, wsin_ref, wcos_ref, wdv_ref, nb_ref,
               emb_ref, wvi_ref, bvi_ref, wvj_ref, bvj_ref,
               v_ref, a_ref, g_ref):
    ang = ang_ref[...]
    v0 = (_mm(jnp.sin(ang), wsin_ref[...]) + _mm(jnp.cos(ang), wcos_ref[...])
          + _mm(dv_ref[...], wdv_ref[...]) + nb_ref[...])
    lab = l_ref[...]
    onehot = (lab == lax.broadcasted_iota(jnp.int32, (_NBV, 21), 1)
              ).astype(jnp.float32)
    v0 = v0 + _mm(onehot, emb_ref[...])
    v_ref[...] = v0
    a_ref[...] = _mm(v0, wvi_ref[...]) + bvi_ref[...]
    g_ref[...] = _mm(v0, wvj_ref[...]) + bvj_ref[...]


def _feat_kernel(ang, dv, lab, wsin, wcos, wdv, nb, emb, wvi, bvi, wvj, bvj):
    ws = [wsin, wcos, wdv, nb, emb, wvi, bvi, wvj, bvj]
    wspecs = [pl.BlockSpec(w.shape, lambda i: (0, 0)) for w in ws]
    return pl.pallas_call(
        _feat_body,
        grid=(_VGRID,),
        in_specs=[
            pl.BlockSpec((_NBV, 24), lambda i: (i, 0)),
            pl.BlockSpec((_NBV, 8), lambda i: (i, 0)),
            pl.BlockSpec((_NBV, 1), lambda i: (i, 0)),
        ] + wspecs,
        out_specs=[pl.BlockSpec((_NBV, DD), lambda i: (i, 0))] * 3,
        out_shape=[jax.ShapeDtypeStruct((NN, DD), jnp.float32)] * 3,
    )(ang, dv, lab, *ws)


def _edge_feat_body(caj_ref, ca_ref, r_ref, cen_ref, ew_ref, eb_ref, out_ref):
    cai = _mm(r_ref[...], ca_ref[...])
    diff = cai - caj_ref[...]
    d2 = jnp.sum(diff * diff, axis=1, keepdims=True)
    d = jnp.sqrt(d2 + 1e-8)
    sigma = (MAXRBF - MINRBF) / NRBF
    z = (d - cen_ref[...]) / sigma
    rbf = jnp.exp(-(z * z))
    out_ref[...] = _mm(rbf, ew_ref[...]) + eb_ref[...]


def _edge_feat_kernel(Caj, Ca16, R, cen, ew, eb):
    return pl.pallas_call(
        _edge_feat_body,
        grid=(_EGRID,),
        in_specs=[
            pl.BlockSpec((_EB, DD), lambda i: (i, 0)),
            pl.BlockSpec((_NBE, DD), lambda i: (i, 0)),
            pl.BlockSpec((_EB, _NBE), lambda i: (0, 0)),
            pl.BlockSpec((1, 16), lambda i: (0, 0)),
            pl.BlockSpec((16, DD), lambda i: (0, 0)),
            pl.BlockSpec((1, DD), lambda i: (0, 0)),
        ],
        out_specs=pl.BlockSpec((_EB, DD), lambda i: (i, 0)),
        out_shape=jax.ShapeDtypeStruct((NN * KK, DD), jnp.float32),
    )(Caj, Ca16, R, cen, ew, eb)


_R_NP = np.kron(np.eye(_NBE, dtype=np.float32), np.ones((KK, 1), np.float32))


def kernel(C, L, chain_idxs, params):
    del chain_idxs
    C0 = C[0]
    Nat, Ca, Cc = C0[:, 0, :], C0[:, 1, :], C0[:, 2, :]
    bv = Ca - Nat
    cv = Cc - Ca
    av = jnp.cross(bv, cv)
    Cb = -0.58273431 * av + 0.56802827 * bv - 0.54067466 * cv + Ca
    dvec = Cb - Ca
    dvec = dvec / (jnp.linalg.norm(dvec, axis=-1, keepdims=True) + 1e-8)
    dvec8 = jnp.pad(dvec, ((0, 0), (0, 5)))

    wlv = jnp.geomspace(MINWL, MAXWL, NWL)
    ang = (Ca[:, :, None] / wlv).reshape(NN, 3 * NWL)

    car = jnp.pad(Ca, ((0, 0), (0, 5)))
    cat3 = jnp.pad(Ca.T, ((0, 5), (0, _NCOL - NN))).reshape(8, _NPAN, _PAN)
    cat3 = jnp.transpose(cat3, (1, 0, 2))
    idx3 = _knn(car, cat3)
    idx = jnp.transpose(idx3, (0, 2, 1)).reshape(NN, KK)
    idx_flat = jnp.concatenate(
        [idx.reshape(-1), jnp.arange(_P - NN * KK, dtype=jnp.int32) % NN])

    p = params
    R = jnp.asarray(_R_NP)
    Rt = R.T
    lab2 = L[0][:, None].astype(jnp.int32)

    nw = p["node_proj"]["w"]
    rows_sin = np.array([d * 16 + w for d in range(3) for w in range(NWL)])
    rows_cos = rows_sin + NWL
    wsin = nw[rows_sin]
    wcos = nw[rows_cos]
    wdv = jnp.pad(nw[48:51], ((0, 5), (0, 0)))
    nb = p["node_proj"]["b"][None, :]
    l0 = p["layers"][0]
    V, A, G = _feat_kernel(
        ang, dvec8, lab2, wsin, wcos, wdv, nb, p["label_embed"],
        l0["w_vi"]["w"], l0["w_vi"]["b"][None], l0["w_vj"]["w"],
        l0["w_vj"]["b"][None])

    Ca16 = jnp.pad(Ca, ((0, 0), (0, DD - 3)))
    Caj = _gather_rows(Ca16, idx_flat, 600)
    cen = jnp.linspace(MINRBF, MAXRBF, NRBF)[None, :]
    E = _edge_feat_kernel(Caj, Ca16, R, cen,
                          p["edge_proj"]["w"], p["edge_proj"]["b"][None])

    nlayers = len(p["layers"])
    for i, ly in enumerate(p["layers"]):
        Gg = _gather_rows(G, idx_flat, 600)
        S = _msg_kernel(E, Gg, A, R, Rt,
                        ly["w_e"]["w"], ly["w_e"]["b"][None],
                        ly["w_m2"]["w"], ly["w_m2"]["b"][None])
        upd = (ly["w_m3"]["w"], ly["w_m3"]["b"][None],
               ly["ln1"]["g"][None], ly["ln1"]["b"][None],
               ly["ffn1"]["w"], ly["ffn1"]["b"][None],
               ly["ffn2"]["w"], ly["ffn2"]["b"][None],
               ly["ln2"]["g"][None], ly["ln2"]["b"][None])
        if i == nlayers - 1:
            (logits,) = _node_upd_kernel(
                V, S, upd,
                [(p["out_proj"]["w"], p["out_proj"]["b"][None])], [NAA])
            break
        nxt = p["layers"][i + 1]
        V, Ae, Ge, A, G = _node_upd_kernel(
            V, S, upd,
            [(ly["we_vi"]["w"], ly["we_vi"]["b"][None]),
             (ly["we_vj"]["w"], ly["we_vj"]["b"][None]),
             (nxt["w_vi"]["w"], nxt["w_vi"]["b"][None]),
             (nxt["w_vj"]["w"], nxt["w_vj"]["b"][None])],
            [DD, DD, DD, DD, DD])
        Gge = _gather_rows(Ge, idx_flat, 600)
        E = _edge_upd_kernel(E, Gge, Ae, R,
                             ly["we_e"]["w"], ly["we_e"]["b"][None],
                             ly["we_2"]["w"], ly["we_2"]["b"][None],
                             ly["ln_e"]["g"][None], ly["ln_e"]["b"][None])

    return logits[None, :, :]


# speedup vs baseline: 4.9612x; 1.3552x over previous
import functools

import numpy as np
import jax
import jax.numpy as jnp
from jax import lax
from jax.experimental import pallas as pl
from jax.experimental.pallas import tpu as pltpu
from jax.experimental.pallas import tpu_sc as plsc

NN = 10000
KK = 30
DD = 128
NRBF = 16
NAA = 20
NWL = 8
MINWL, MAXWL = 3.5, 12.0
MINRBF, MAXRBF = 2.0, 22.0

_RB = 200
_NPAN = 10
_PAN = 1024
_NCOL = _NPAN * _PAN

_NBE = 80
_EB = _NBE * KK
_EGRID = NN // _NBE

_NBV = 1000
_VGRID = NN // _NBV

_P = 307200
_NW = 32
_PERW = _P // _NW


def _gelu(x):
    return jax.nn.gelu(x)


def _mm(a, b):
    return lax.dot_general(a, b, (((1,), (0,)), ((), ())),
                           preferred_element_type=jnp.float32)


def _ln(x, g, b):
    mu = jnp.mean(x, axis=-1, keepdims=True)
    var = jnp.mean((x - mu) ** 2, axis=-1, keepdims=True)
    return (x - mu) * lax.rsqrt(var + 1e-5) * g + b


def _knn_body(car_ref, cat_ref, idx_ref, vals_ref):
    car = car_ref[...]
    x2r = ((car[:, 0:1] * car[:, 0:1] + car[:, 2:3] * car[:, 2:3])
           + car[:, 1:2] * car[:, 1:2])
    carb = car.astype(jnp.bfloat16).astype(jnp.float32)

    def init(p, _):
        catp = cat_ref[p]
        x2c = ((catp[0:1, :] * catp[0:1, :] + catp[2:3, :] * catp[2:3, :])
               + catp[1:2, :] * catp[1:2, :])
        catb = catp.astype(jnp.bfloat16).astype(jnp.float32)
        p0 = carb[:, 0:1] * catb[0:1, :]
        p1 = carb[:, 1:2] * catb[1:2, :]
        p2 = carb[:, 2:3] * catb[2:3, :]
        s1 = p0 + p1
        bp = s1 - p0
        e1 = (p0 - (s1 - bp)) + (p1 - bp)
        s2 = s1 + p2
        bp2 = s2 - s1
        e2 = (s1 - (s2 - bp2)) + (p2 - bp2)
        dot = s2 + (e1 + e2)
        col = lax.broadcasted_iota(jnp.int32, (_RB, _PAN), 1) + p * _PAN
        d2 = x2r + x2c - 2.0 * dot
        vals_ref[p] = jnp.where(col >= NN, 1e30, d2)
        return _

    lax.fori_loop(0, _NPAN, init, 0)

    srow = lax.broadcasted_iota(jnp.int32, (1, KK, _RB), 1)

    def extract(j, _):
        def pmin(p, m):
            return jnp.minimum(m, jnp.min(vals_ref[p], axis=1, keepdims=True))
        m = lax.fori_loop(0, _NPAN, pmin, jnp.full((_RB, 1), 1e30, jnp.float32))

        def pargmin(p, best):
            v = vals_ref[p]
            col = lax.broadcasted_iota(jnp.int32, (_RB, _PAN), 1) + p * _PAN
            cand = jnp.where(v == m, col, jnp.int32(2 ** 30))
            return jnp.minimum(best, jnp.min(cand, axis=1))
        sel = lax.fori_loop(0, _NPAN, pargmin,
                            jnp.full((_RB,), 2 ** 30, jnp.int32))

        idx_ref[...] = jnp.where(srow == j, sel[None, None, :], idx_ref[...])

        def pupd(p, _c):
            v = vals_ref[p]
            col = lax.broadcasted_iota(jnp.int32, (_RB, _PAN), 1) + p * _PAN
            vals_ref[p] = jnp.where(col == sel[:, None], 1e30, v)
            return _c
        lax.fori_loop(0, _NPAN, pupd, 0)
        return _

    lax.fori_loop(0, KK, extract, 0)


def _knn(car, cat3):
    return pl.pallas_call(
        _knn_body,
        grid=(NN // _RB,),
        in_specs=[
            pl.BlockSpec((_RB, 8), lambda i: (i, 0)),
            pl.BlockSpec((_NPAN, 8, _PAN), lambda i: (0, 0, 0)),
        ],
        out_specs=pl.BlockSpec((1, KK, _RB), lambda i: (i, 0, 0)),
        out_shape=jax.ShapeDtypeStruct((NN // _RB, KK, _RB), jnp.int32),
        scratch_shapes=[pltpu.VMEM((_NPAN, _RB, _PAN), jnp.float32)],
    )(car, cat3)


@functools.lru_cache(maxsize=None)
def _sc_gather(dt, chunk):
    mesh = plsc.VectorSubcoreMesh(core_axis_name="c", subcore_axis_name="s")
    nit = _PERW // chunk

    @functools.partial(
        pl.kernel, mesh=mesh,
        out_type=jax.ShapeDtypeStruct((_P, dt), jnp.float32),
        scratch_types=[
            pltpu.VMEM((chunk,), jnp.int32),
            pltpu.VMEM((chunk, dt), jnp.float32),
            pltpu.SemaphoreType.DMA,
        ],
    )
    def gk(table_hbm, idx_hbm, out_hbm, idx_v, rows_v, sem):
        wid = lax.axis_index("s") * 2 + lax.axis_index("c")
        base = wid * _PERW

        def body(t, carry):
            off = base + t * chunk
            pltpu.sync_copy(idx_hbm.at[pl.ds(off, chunk)], idx_v)
            pltpu.async_copy(table_hbm.at[idx_v], rows_v, sem).wait()
            pltpu.sync_copy(rows_v, out_hbm.at[pl.ds(off, chunk)])
            return carry

        lax.fori_loop(0, nit, body, 0)

    return gk


def _gather_rows(table, idx_pad, chunk):
    return _sc_gather(table.shape[1], chunk)(table, idx_pad)


def _msg_body(e_ref, gg_ref, a_ref, r_ref, rt_ref,
              we_ref, be_ref, w2_ref, b2_ref, out_ref):
    pre = (_mm(r_ref[...], a_ref[...]) + gg_ref[...]
           + _mm(e_ref[...], we_ref[...]) + be_ref[...])
    m = _gelu(pre)
    y = _gelu(_mm(m, w2_ref[...]) + b2_ref[...])
    out_ref[...] = _mm(rt_ref[...], y) * (1.0 / KK)


def _msg_kernel(E, Gg, A, R, Rt, we, be, w2, b2):
    return pl.pallas_call(
        _msg_body,
        grid=(_EGRID,),
        in_specs=[
            pl.BlockSpec((_EB, DD), lambda i: (i, 0)),
            pl.BlockSpec((_EB, DD), lambda i: (i, 0)),
            pl.BlockSpec((_NBE, DD), lambda i: (i, 0)),
            pl.BlockSpec((_EB, _NBE), lambda i: (0, 0)),
            pl.BlockSpec((_NBE, _EB), lambda i: (0, 0)),
            pl.BlockSpec((DD, DD), lambda i: (0, 0)),
            pl.BlockSpec((1, DD), lambda i: (0, 0)),
            pl.BlockSpec((DD, DD), lambda i: (0, 0)),
            pl.BlockSpec((1, DD), lambda i: (0, 0)),
        ],
        out_specs=pl.BlockSpec((_NBE, DD), lambda i: (i, 0)),
        out_shape=jax.ShapeDtypeStruct((NN, DD), jnp.float32),
    )(E, Gg, A, R, Rt, we, be, w2, b2)


def _edge_upd_body(e_ref, gg_ref, a_ref, r_ref,
                   we_ref, be_ref, w2_ref, b2_ref, g_ref, gb_ref, out_ref):
    pre = (_mm(r_ref[...], a_ref[...]) + gg_ref[...]
           + _mm(e_ref[...], we_ref[...]) + be_ref[...])
    h = _mm(_gelu(pre), w2_ref[...]) + b2_ref[...]
    out_ref[...] = _ln(e_ref[...] + h, g_ref[...], gb_ref[...])


def _edge_upd_kernel(E, Gg, A, R, we, be, w2, b2, g, gb):
    return pl.pallas_call(
        _edge_upd_body,
        grid=(_EGRID,),
        in_specs=[
            pl.BlockSpec((_EB, DD), lambda i: (i, 0)),
            pl.BlockSpec((_EB, DD), lambda i: (i, 0)),
            pl.BlockSpec((_NBE, DD), lambda i: (i, 0)),
            pl.BlockSpec((_EB, _NBE), lambda i: (0, 0)),
            pl.BlockSpec((DD, DD), lambda i: (0, 0)),
            pl.BlockSpec((1, DD), lambda i: (0, 0)),
            pl.BlockSpec((DD, DD), lambda i: (0, 0)),
            pl.BlockSpec((1, DD), lambda i: (0, 0)),
            pl.BlockSpec((1, DD), lambda i: (0, 0)),
            pl.BlockSpec((1, DD), lambda i: (0, 0)),
        ],
        out_specs=pl.BlockSpec((_EB, DD), lambda i: (i, 0)),
        out_shape=jax.ShapeDtypeStruct((NN * KK, DD), jnp.float32),
    )(E, Gg, A, R, we, be, w2, b2, g, gb)


def _node_upd_body(nouts, v_ref, s_ref, w3_ref, b3_ref, g1_ref, gb1_ref,
                   f1_ref, fb1_ref, f2_ref, fb2_ref, g2_ref, gb2_ref,
                   *rest):
    u = _ln(v_ref[...] + _mm(s_ref[...], w3_ref[...]) + b3_ref[...],
            g1_ref[...], gb1_ref[...])
    h = _mm(_gelu(_mm(u, f1_ref[...]) + fb1_ref[...]), f2_ref[...]) + fb2_ref[...]
    v2 = _ln(u + h, g2_ref[...], gb2_ref[...])
    wrefs = rest[:-nouts]
    orefs = rest[-nouts:]
    if nouts == 1:
        ow, ob = wrefs
        orefs[0][...] = _mm(v2, ow[...]) + ob[...]
    else:
        orefs[0][...] = v2
        for t in range(nouts - 1):
            w, b = wrefs[2 * t], wrefs[2 * t + 1]
            orefs[t + 1][...] = _mm(v2, w[...]) + b[...]


def _node_upd_kernel(V, S, upd_params, extra_ws, out_dims):
    proj_only = len(out_dims) == 1 and out_dims[0] != DD
    nouts = len(out_dims)
    win = list(upd_params)
    for w, b in extra_ws:
        win += [w, b]
    wspecs = []
    for w in win:
        wspecs.append(pl.BlockSpec(w.shape, lambda i: (0,) * w.ndim))
    out_specs = [pl.BlockSpec((_NBV, d), lambda i: (i, 0)) for d in out_dims]
    out_shape = [jax.ShapeDtypeStruct((NN, d), jnp.float32) for d in out_dims]
    return pl.pallas_call(
        functools.partial(_node_upd_body, nouts),
        grid=(_VGRID,),
        in_specs=[
            pl.BlockSpec((_NBV, DD), lambda i: (i, 0)),
            pl.BlockSpec((_NBV, DD), lambda i: (i, 0)),
        ] + wspecs,
        out_specs=out_specs,
        out_shape=out_shape,
    )(V, S, *win)


def _feat_body(ang_ref, dv_ref, l_ref, wsin_ref, wcos_ref, wdv_ref, nb_ref,
               emb_ref, wvi_ref, bvi_ref, wvj_ref, bvj_ref,
               v_ref, a_ref, g_ref):
    ang = ang_ref[...]
    v0 = (_mm(jnp.sin(ang), wsin_ref[...]) + _mm(jnp.cos(ang), wcos_ref[...])
          + _mm(dv_ref[...], wdv_ref[...]) + nb_ref[...])
    lab = l_ref[...]
    onehot = (lab == lax.broadcasted_iota(jnp.int32, (_NBV, 21), 1)
              ).astype(jnp.float32)
    v0 = v0 + _mm(onehot, emb_ref[...])
    v_ref[...] = v0
    a_ref[...] = _mm(v0, wvi_ref[...]) + bvi_ref[...]
    g_ref[...] = _mm(v0, wvj_ref[...]) + bvj_ref[...]


def _feat_kernel(ang, dv, lab, wsin, wcos, wdv, nb, emb, wvi, bvi, wvj, bvj):
    ws = [wsin, wcos, wdv, nb, emb, wvi, bvi, wvj, bvj]
    wspecs = [pl.BlockSpec(w.shape, lambda i: (0, 0)) for w in ws]
    return pl.pallas_call(
        _feat_body,
        grid=(_VGRID,),
        in_specs=[
            pl.BlockSpec((_NBV, 24), lambda i: (i, 0)),
            pl.BlockSpec((_NBV, 8), lambda i: (i, 0)),
            pl.BlockSpec((_NBV, 1), lambda i: (i, 0)),
        ] + wspecs,
        out_specs=[pl.BlockSpec((_NBV, DD), lambda i: (i, 0))] * 3,
        out_shape=[jax.ShapeDtypeStruct((NN, DD), jnp.float32)] * 3,
    )(ang, dv, lab, *ws)


def _edge_feat_body(caj_ref, ca_ref, r_ref, cen_ref, ew_ref, eb_ref, out_ref):
    cai = _mm(r_ref[...], ca_ref[...])
    diff = cai - caj_ref[...]
    d2 = jnp.sum(diff * diff, axis=1, keepdims=True)
    d = jnp.sqrt(d2 + 1e-8)
    sigma = (MAXRBF - MINRBF) / NRBF
    z = (d - cen_ref[...]) / sigma
    rbf = jnp.exp(-(z * z))
    out_ref[...] = _mm(rbf, ew_ref[...]) + eb_ref[...]


def _edge_feat_kernel(Caj, Ca16, R, cen, ew, eb):
    return pl.pallas_call(
        _edge_feat_body,
        grid=(_EGRID,),
        in_specs=[
            pl.BlockSpec((_EB, DD), lambda i: (i, 0)),
            pl.BlockSpec((_NBE, DD), lambda i: (i, 0)),
            pl.BlockSpec((_EB, _NBE), lambda i: (0, 0)),
            pl.BlockSpec((1, 16), lambda i: (0, 0)),
            pl.BlockSpec((16, DD), lambda i: (0, 0)),
            pl.BlockSpec((1, DD), lambda i: (0, 0)),
        ],
        out_specs=pl.BlockSpec((_EB, DD), lambda i: (i, 0)),
        out_shape=jax.ShapeDtypeStruct((NN * KK, DD), jnp.float32),
    )(Caj, Ca16, R, cen, ew, eb)


_R_NP = np.kron(np.eye(_NBE, dtype=np.float32), np.ones((KK, 1), np.float32))


def kernel(C, L, chain_idxs, params):
    del chain_idxs
    C0 = C[0]
    Nat, Ca, Cc = C0[:, 0, :], C0[:, 1, :], C0[:, 2, :]
    bv = Ca - Nat
    cv = Cc - Ca
    av = jnp.cross(bv, cv)
    Cb = -0.58273431 * av + 0.56802827 * bv - 0.54067466 * cv + Ca
    dvec = Cb - Ca
    dvec = dvec / (jnp.linalg.norm(dvec, axis=-1, keepdims=True) + 1e-8)
    dvec8 = jnp.pad(dvec, ((0, 0), (0, 5)))

    wlv = jnp.geomspace(MINWL, MAXWL, NWL)
    ang = (Ca[:, :, None] / wlv).reshape(NN, 3 * NWL)

    car = jnp.pad(Ca, ((0, 0), (0, 5)))
    cat3 = jnp.pad(Ca.T, ((0, 5), (0, _NCOL - NN))).reshape(8, _NPAN, _PAN)
    cat3 = jnp.transpose(cat3, (1, 0, 2))
    idx3 = _knn(car, cat3)
    idx = jnp.transpose(idx3, (0, 2, 1)).reshape(NN, KK)
    idx_flat = jnp.concatenate(
        [idx.reshape(-1), jnp.arange(_P - NN * KK, dtype=jnp.int32) % NN])

    p = params
    R = jnp.asarray(_R_NP)
    Rt = R.T
    lab2 = L[0][:, None].astype(jnp.int32)

    nw = p["node_proj"]["w"]
    rows_sin = np.array([d * 16 + w for d in range(3) for w in range(NWL)])
    rows_cos = rows_sin + NWL
    wsin = nw[rows_sin]
    wcos = nw[rows_cos]
    wdv = jnp.pad(nw[48:51], ((0, 5), (0, 0)))
    nb = p["node_proj"]["b"][None, :]
    l0 = p["layers"][0]
    V, A, G = _feat_kernel(
        ang, dvec8, lab2, wsin, wcos, wdv, nb, p["label_embed"],
        l0["w_vi"]["w"], l0["w_vi"]["b"][None], l0["w_vj"]["w"],
        l0["w_vj"]["b"][None])

    Ca16 = jnp.pad(Ca, ((0, 0), (0, DD - 3)))
    Caj = _gather_rows(Ca16, idx_flat, 600)
    cen = jnp.linspace(MINRBF, MAXRBF, NRBF)[None, :]
    E = _edge_feat_kernel(Caj, Ca16, R, cen,
                          p["edge_proj"]["w"], p["edge_proj"]["b"][None])

    nlayers = len(p["layers"])
    for i, ly in enumerate(p["layers"]):
        Gg = _gather_rows(G, idx_flat, 600)
        S = _msg_kernel(E, Gg, A, R, Rt,
                        ly["w_e"]["w"], ly["w_e"]["b"][None],
                        ly["w_m2"]["w"], ly["w_m2"]["b"][None])
        upd = (ly["w_m3"]["w"], ly["w_m3"]["b"][None],
               ly["ln1"]["g"][None], ly["ln1"]["b"][None],
               ly["ffn1"]["w"], ly["ffn1"]["b"][None],
               ly["ffn2"]["w"], ly["ffn2"]["b"][None],
               ly["ln2"]["g"][None], ly["ln2"]["b"][None])
        if i == nlayers - 1:
            (logits,) = _node_upd_kernel(
                V, S, upd,
                [(p["out_proj"]["w"], p["out_proj"]["b"][None])], [NAA])
            break
        nxt = p["layers"][i + 1]
        V, Ae, Ge, A, G = _node_upd_kernel(
            V, S, upd,
            [(ly["we_vi"]["w"], ly["we_vi"]["b"][None]),
             (ly["we_vj"]["w"], ly["we_vj"]["b"][None]),
             (nxt["w_vi"]["w"], nxt["w_vi"]["b"][None]),
             (nxt["w_vj"]["w"], nxt["w_vj"]["b"][None])],
            [DD, DD, DD, DD, DD])
        Gge = _gather_rows(Ge, idx_flat, 600)
        E = _edge_upd_kernel(E, Gge, Ae, R,
                             ly["we_e"]["w"], ly["we_e"]["b"][None],
                             ly["we_2"]["w"], ly["we_2"]["b"][None],
                             ly["ln_e"]["g"][None], ly["ln_e"]["b"][None])

    return logits[None, :, :]


# speedup vs baseline: 10.5990x; 2.1364x over previous
import functools

import numpy as np
import jax
import jax.numpy as jnp
from jax import lax
from jax.experimental import pallas as pl
from jax.experimental.pallas import tpu as pltpu
from jax.experimental.pallas import tpu_sc as plsc

NN = 10000
KK = 30
DD = 128
NRBF = 16
NAA = 20
NWL = 8
MINWL, MAXWL = 3.5, 12.0
MINRBF, MAXRBF = 2.0, 22.0

_RB = 200
_NPAN = 10
_PAN = 1024
_NCOL = _NPAN * _PAN

_NBE = 80
_EB = _NBE * KK
_EGRID = NN // _NBE

_NBV = 1000
_VGRID = NN // _NBV

_P = 307200
_NW = 32
_PERW = _P // _NW


def _gelu(x):
    return jax.nn.gelu(x)


def _mm(a, b):
    return lax.dot_general(a, b, (((1,), (0,)), ((), ())),
                           preferred_element_type=jnp.float32)


def _ln(x, g, b):
    mu = jnp.mean(x, axis=-1, keepdims=True)
    var = jnp.mean((x - mu) ** 2, axis=-1, keepdims=True)
    return (x - mu) * lax.rsqrt(var + 1e-5) * g + b


_NSUB = _PAN // 128
_TCA = 6


def _knn_body(car_ref, cat_ref, idx_ref, vals_ref, cv_ref, cc_ref):
    car = car_ref[...]
    x2r = ((car[:, 0:1] * car[:, 0:1] + car[:, 2:3] * car[:, 2:3])
           + car[:, 1:2] * car[:, 1:2])
    carb = car.astype(jnp.bfloat16).astype(jnp.float32)

    def init_vals():
        def init(p, _):
            catp = cat_ref[p]
            x2c = ((catp[0:1, :] * catp[0:1, :] + catp[2:3, :] * catp[2:3, :])
                   + catp[1:2, :] * catp[1:2, :])
            catb = catp.astype(jnp.bfloat16).astype(jnp.float32)
            p0 = carb[:, 0:1] * catb[0:1, :]
            p1 = carb[:, 1:2] * catb[1:2, :]
            p2 = carb[:, 2:3] * catb[2:3, :]
            s1 = p0 + p1
            bp = s1 - p0
            e1 = (p0 - (s1 - bp)) + (p1 - bp)
            s2 = s1 + p2
            bp2 = s2 - s1
            e2 = (s1 - (s2 - bp2)) + (p2 - bp2)
            dot = s2 + (e1 + e2)
            col = lax.broadcasted_iota(jnp.int32, (_RB, _PAN), 1) + p * _PAN
            d2 = x2r + x2c - 2.0 * dot
            vals_ref[p] = jnp.where(col >= NN, 1e30, d2)
            return _
        lax.fori_loop(0, _NPAN, init, 0)

    init_vals()

    laneio = lax.broadcasted_iota(jnp.int32, (_RB, 128), 1)
    srow = lax.broadcasted_iota(jnp.int32, (1, KK, _RB), 1)

    def cache_t(t, _):
        def scan_p(p, MC):
            M, Cc = MC
            for c in range(_NSUB):
                v = vals_ref[p, :, pl.ds(c * 128, 128)]
                colv = laneio + (p * _PAN + c * 128)
                better = v < M
                M = jnp.where(better, v, M)
                Cc = jnp.where(better, colv, Cc)
            return (M, Cc)
        M, Cc = lax.fori_loop(0, _NPAN, scan_p,
                              (jnp.full((_RB, 128), 1e30, jnp.float32),
                               jnp.full((_RB, 128), 2 ** 30, jnp.int32)))
        cv_ref[t] = M
        cc_ref[t] = Cc

        def mask_p(p, _c):
            for c in range(_NSUB):
                v = vals_ref[p, :, pl.ds(c * 128, 128)]
                colv = laneio + (p * _PAN + c * 128)
                vals_ref[p, :, pl.ds(c * 128, 128)] = jnp.where(
                    colv == Cc, 1e30, v)
            return _c
        lax.fori_loop(0, _NPAN, mask_p, 0)
        return _
    lax.fori_loop(0, _TCA, cache_t, 0)

    def extract(j, selcnt):
        mm = jnp.full((_RB, 1), 1e30, jnp.float32)
        for t in range(_TCA):
            mm = jnp.minimum(mm, jnp.min(cv_ref[t], axis=1, keepdims=True))
        sel = jnp.full((_RB,), 2 ** 30, jnp.int32)
        for t in range(_TCA):
            cand = jnp.where(cv_ref[t] == mm, cc_ref[t], jnp.int32(2 ** 30))
            sel = jnp.minimum(sel, jnp.min(cand, axis=1))
        idx_ref[...] = jnp.where(srow == j, sel[None, None, :], idx_ref[...])
        for t in range(_TCA):
            cv_ref[t] = jnp.where(cc_ref[t] == sel[:, None], 1e30, cv_ref[t])
        lane = lax.rem(sel, jnp.int32(128))
        return selcnt + jnp.where(laneio == lane[:, None], 1, 0)

    selcnt = lax.fori_loop(0, KK, extract,
                           jnp.zeros((_RB, 128), jnp.int32))

    @pl.when(jnp.any(selcnt >= _TCA))
    def _fallback():
        init_vals()

        def extract_full(j, _):
            def pmin(p, m):
                return jnp.minimum(
                    m, jnp.min(vals_ref[p], axis=1, keepdims=True))
            m = lax.fori_loop(0, _NPAN, pmin,
                              jnp.full((_RB, 1), 1e30, jnp.float32))

            def pargmin(p, best):
                v = vals_ref[p]
                col = lax.broadcasted_iota(jnp.int32, (_RB, _PAN), 1) + p * _PAN
                cand = jnp.where(v == m, col, jnp.int32(2 ** 30))
                return jnp.minimum(best, jnp.min(cand, axis=1))
            sel = lax.fori_loop(0, _NPAN, pargmin,
                                jnp.full((_RB,), 2 ** 30, jnp.int32))

            idx_ref[...] = jnp.where(srow == j, sel[None, None, :],
                                     idx_ref[...])

            def pupd(p, _c):
                v = vals_ref[p]
                col = lax.broadcasted_iota(jnp.int32, (_RB, _PAN), 1) + p * _PAN
                vals_ref[p] = jnp.where(col == sel[:, None], 1e30, v)
                return _c
            lax.fori_loop(0, _NPAN, pupd, 0)
            return _

        lax.fori_loop(0, KK, extract_full, 0)


def _knn(car, cat3):
    return pl.pallas_call(
        _knn_body,
        grid=(NN // _RB,),
        in_specs=[
            pl.BlockSpec((_RB, 8), lambda i: (i, 0)),
            pl.BlockSpec((_NPAN, 8, _PAN), lambda i: (0, 0, 0)),
        ],
        out_specs=pl.BlockSpec((1, KK, _RB), lambda i: (i, 0, 0)),
        out_shape=jax.ShapeDtypeStruct((NN // _RB, KK, _RB), jnp.int32),
        scratch_shapes=[pltpu.VMEM((_NPAN, _RB, _PAN), jnp.float32),
                        pltpu.VMEM((_TCA, _RB, 128), jnp.float32),
                        pltpu.VMEM((_TCA, _RB, 128), jnp.int32)],
    )(car, cat3)


@functools.lru_cache(maxsize=None)
def _sc_gather(dt, chunk):
    mesh = plsc.VectorSubcoreMesh(core_axis_name="c", subcore_axis_name="s")
    nit = _PERW // chunk

    @functools.partial(
        pl.kernel, mesh=mesh,
        out_type=jax.ShapeDtypeStruct((_P, dt), jnp.float32),
        scratch_types=[
            pltpu.VMEM((chunk,), jnp.int32),
            pltpu.VMEM((chunk, dt), jnp.float32),
            pltpu.SemaphoreType.DMA,
        ],
    )
    def gk(table_hbm, idx_hbm, out_hbm, idx_v, rows_v, sem):
        wid = lax.axis_index("s") * 2 + lax.axis_index("c")
        base = wid * _PERW

        def body(t, carry):
            off = base + t * chunk
            pltpu.sync_copy(idx_hbm.at[pl.ds(off, chunk)], idx_v)
            pltpu.async_copy(table_hbm.at[idx_v], rows_v, sem).wait()
            pltpu.sync_copy(rows_v, out_hbm.at[pl.ds(off, chunk)])
            return carry

        lax.fori_loop(0, nit, body, 0)

    return gk


def _gather_rows(table, idx_pad, chunk):
    return _sc_gather(table.shape[1], chunk)(table, idx_pad)


def _msg_body(e_ref, gg_ref, a_ref, r_ref, rt_ref,
              we_ref, be_ref, w2_ref, b2_ref, out_ref):
    pre = (_mm(r_ref[...], a_ref[...]) + gg_ref[...]
           + _mm(e_ref[...], we_ref[...]) + be_ref[...])
    m = _gelu(pre)
    y = _gelu(_mm(m, w2_ref[...]) + b2_ref[...])
    out_ref[...] = _mm(rt_ref[...], y) * (1.0 / KK)


def _msg_kernel(E, Gg, A, R, Rt, we, be, w2, b2):
    return pl.pallas_call(
        _msg_body,
        grid=(_EGRID,),
        in_specs=[
            pl.BlockSpec((_EB, DD), lambda i: (i, 0)),
            pl.BlockSpec((_EB, DD), lambda i: (i, 0)),
            pl.BlockSpec((_NBE, DD), lambda i: (i, 0)),
            pl.BlockSpec((_EB, _NBE), lambda i: (0, 0)),
            pl.BlockSpec((_NBE, _EB), lambda i: (0, 0)),
            pl.BlockSpec((DD, DD), lambda i: (0, 0)),
            pl.BlockSpec((1, DD), lambda i: (0, 0)),
            pl.BlockSpec((DD, DD), lambda i: (0, 0)),
            pl.BlockSpec((1, DD), lambda i: (0, 0)),
        ],
        out_specs=pl.BlockSpec((_NBE, DD), lambda i: (i, 0)),
        out_shape=jax.ShapeDtypeStruct((NN, DD), jnp.float32),
    )(E, Gg, A, R, Rt, we, be, w2, b2)


def _edge_upd_body(e_ref, gg_ref, a_ref, r_ref,
                   we_ref, be_ref, w2_ref, b2_ref, g_ref, gb_ref, out_ref):
    pre = (_mm(r_ref[...], a_ref[...]) + gg_ref[...]
           + _mm(e_ref[...], we_ref[...]) + be_ref[...])
    h = _mm(_gelu(pre), w2_ref[...]) + b2_ref[...]
    out_ref[...] = _ln(e_ref[...] + h, g_ref[...], gb_ref[...])


def _edge_upd_kernel(E, Gg, A, R, we, be, w2, b2, g, gb):
    return pl.pallas_call(
        _edge_upd_body,
        grid=(_EGRID,),
        in_specs=[
            pl.BlockSpec((_EB, DD), lambda i: (i, 0)),
            pl.BlockSpec((_EB, DD), lambda i: (i, 0)),
            pl.BlockSpec((_NBE, DD), lambda i: (i, 0)),
            pl.BlockSpec((_EB, _NBE), lambda i: (0, 0)),
            pl.BlockSpec((DD, DD), lambda i: (0, 0)),
            pl.BlockSpec((1, DD), lambda i: (0, 0)),
            pl.BlockSpec((DD, DD), lambda i: (0, 0)),
            pl.BlockSpec((1, DD), lambda i: (0, 0)),
            pl.BlockSpec((1, DD), lambda i: (0, 0)),
            pl.BlockSpec((1, DD), lambda i: (0, 0)),
        ],
        out_specs=pl.BlockSpec((_EB, DD), lambda i: (i, 0)),
        out_shape=jax.ShapeDtypeStruct((NN * KK, DD), jnp.float32),
    )(E, Gg, A, R, we, be, w2, b2, g, gb)


def _node_upd_body(nouts, v_ref, s_ref, w3_ref, b3_ref, g1_ref, gb1_ref,
                   f1_ref, fb1_ref, f2_ref, fb2_ref, g2_ref, gb2_ref,
                   *rest):
    u = _ln(v_ref[...] + _mm(s_ref[...], w3_ref[...]) + b3_ref[...],
            g1_ref[...], gb1_ref[...])
    h = _mm(_gelu(_mm(u, f1_ref[...]) + fb1_ref[...]), f2_ref[...]) + fb2_ref[...]
    v2 = _ln(u + h, g2_ref[...], gb2_ref[...])
    wrefs = rest[:-nouts]
    orefs = rest[-nouts:]
    if nouts == 1:
        ow, ob = wrefs
        orefs[0][...] = _mm(v2, ow[...]) + ob[...]
    else:
        orefs[0][...] = v2
        for t in range(nouts - 1):
            w, b = wrefs[2 * t], wrefs[2 * t + 1]
            orefs[t + 1][...] = _mm(v2, w[...]) + b[...]


def _node_upd_kernel(V, S, upd_params, extra_ws, out_dims):
    proj_only = len(out_dims) == 1 and out_dims[0] != DD
    nouts = len(out_dims)
    win = list(upd_params)
    for w, b in extra_ws:
        win += [w, b]
    wspecs = []
    for w in win:
        wspecs.append(pl.BlockSpec(w.shape, lambda i: (0,) * w.ndim))
    out_specs = [pl.BlockSpec((_NBV, d), lambda i: (i, 0)) for d in out_dims]
    out_shape = [jax.ShapeDtypeStruct((NN, d), jnp.float32) for d in out_dims]
    return pl.pallas_call(
        functools.partial(_node_upd_body, nouts),
        grid=(_VGRID,),
        in_specs=[
            pl.BlockSpec((_NBV, DD), lambda i: (i, 0)),
            pl.BlockSpec((_NBV, DD), lambda i: (i, 0)),
        ] + wspecs,
        out_specs=out_specs,
        out_shape=out_shape,
    )(V, S, *win)


def _feat_body(ang_ref, dv_ref, l_ref, wsin_ref, wcos_ref, wdv_ref, nb_ref,
               emb_ref, wvi_ref, bvi_ref, wvj_ref, bvj_ref,
               v_ref, a_ref, g_ref):
    ang = ang_ref[...]
    v0 = (_mm(jnp.sin(ang), wsin_ref[...]) + _mm(jnp.cos(ang), wcos_ref[...])
          + _mm(dv_ref[...], wdv_ref[...]) + nb_ref[...])
    lab = l_ref[...]
    onehot = (lab == lax.broadcasted_iota(jnp.int32, (_NBV, 21), 1)
              ).astype(jnp.float32)
    v0 = v0 + _mm(onehot, emb_ref[...])
    v_ref[...] = v0
    a_ref[...] = _mm(v0, wvi_ref[...]) + bvi_ref[...]
    g_ref[...] = _mm(v0, wvj_ref[...]) + bvj_ref[...]


def _feat_kernel(ang, dv, lab, wsin, wcos, wdv, nb, emb, wvi, bvi, wvj, bvj):
    ws = [wsin, wcos, wdv, nb, emb, wvi, bvi, wvj, bvj]
    wspecs = [pl.BlockSpec(w.shape, lambda i: (0, 0)) for w in ws]
    return pl.pallas_call(
        _feat_body,
        grid=(_VGRID,),
        in_specs=[
            pl.BlockSpec((_NBV, 24), lambda i: (i, 0)),
            pl.BlockSpec((_NBV, 8), lambda i: (i, 0)),
            pl.BlockSpec((_NBV, 1), lambda i: (i, 0)),
        ] + wspecs,
        out_specs=[pl.BlockSpec((_NBV, DD), lambda i: (i, 0))] * 3,
        out_shape=[jax.ShapeDtypeStruct((NN, DD), jnp.float32)] * 3,
    )(ang, dv, lab, *ws)


def _edge_feat_body(caj_ref, ca_ref, r_ref, cen_ref, ew_ref, eb_ref, out_ref):
    cai = _mm(r_ref[...], ca_ref[...])
    diff = cai - caj_ref[...]
    d2 = jnp.sum(diff * diff, axis=1, keepdims=True)
    d = jnp.sqrt(d2 + 1e-8)
    sigma = (MAXRBF - MINRBF) / NRBF
    z = (d - cen_ref[...]) / sigma
    rbf = jnp.exp(-(z * z))
    out_ref[...] = _mm(rbf, ew_ref[...]) + eb_ref[...]


def _edge_feat_kernel(Caj, Ca16, R, cen, ew, eb):
    return pl.pallas_call(
        _edge_feat_body,
        grid=(_EGRID,),
        in_specs=[
            pl.BlockSpec((_EB, DD), lambda i: (i, 0)),
            pl.BlockSpec((_NBE, DD), lambda i: (i, 0)),
            pl.BlockSpec((_EB, _NBE), lambda i: (0, 0)),
            pl.BlockSpec((1, 16), lambda i: (0, 0)),
            pl.BlockSpec((16, DD), lambda i: (0, 0)),
            pl.BlockSpec((1, DD), lambda i: (0, 0)),
        ],
        out_specs=pl.BlockSpec((_EB, DD), lambda i: (i, 0)),
        out_shape=jax.ShapeDtypeStruct((NN * KK, DD), jnp.float32),
    )(Caj, Ca16, R, cen, ew, eb)


_R_NP = np.kron(np.eye(_NBE, dtype=np.float32), np.ones((KK, 1), np.float32))


def kernel(C, L, chain_idxs, params):
    del chain_idxs
    C0 = C[0]
    Nat, Ca, Cc = C0[:, 0, :], C0[:, 1, :], C0[:, 2, :]
    bv = Ca - Nat
    cv = Cc - Ca
    av = jnp.cross(bv, cv)
    Cb = -0.58273431 * av + 0.56802827 * bv - 0.54067466 * cv + Ca
    dvec = Cb - Ca
    dvec = dvec / (jnp.linalg.norm(dvec, axis=-1, keepdims=True) + 1e-8)
    dvec8 = jnp.pad(dvec, ((0, 0), (0, 5)))

    wlv = jnp.geomspace(MINWL, MAXWL, NWL)
    ang = (Ca[:, :, None] / wlv).reshape(NN, 3 * NWL)

    car = jnp.pad(Ca, ((0, 0), (0, 5)))
    cat3 = jnp.pad(Ca.T, ((0, 5), (0, _NCOL - NN))).reshape(8, _NPAN, _PAN)
    cat3 = jnp.transpose(cat3, (1, 0, 2))
    idx3 = _knn(car, cat3)
    idx = jnp.transpose(idx3, (0, 2, 1)).reshape(NN, KK)
    idx_flat = jnp.concatenate(
        [idx.reshape(-1), jnp.arange(_P - NN * KK, dtype=jnp.int32) % NN])

    p = params
    R = jnp.asarray(_R_NP)
    Rt = R.T
    lab2 = L[0][:, None].astype(jnp.int32)

    nw = p["node_proj"]["w"]
    rows_sin = np.array([d * 16 + w for d in range(3) for w in range(NWL)])
    rows_cos = rows_sin + NWL
    wsin = nw[rows_sin]
    wcos = nw[rows_cos]
    wdv = jnp.pad(nw[48:51], ((0, 5), (0, 0)))
    nb = p["node_proj"]["b"][None, :]
    l0 = p["layers"][0]
    V, A, G = _feat_kernel(
        ang, dvec8, lab2, wsin, wcos, wdv, nb, p["label_embed"],
        l0["w_vi"]["w"], l0["w_vi"]["b"][None], l0["w_vj"]["w"],
        l0["w_vj"]["b"][None])

    Ca16 = jnp.pad(Ca, ((0, 0), (0, DD - 3)))
    Caj = _gather_rows(Ca16, idx_flat, 600)
    cen = jnp.linspace(MINRBF, MAXRBF, NRBF)[None, :]
    E = _edge_feat_kernel(Caj, Ca16, R, cen,
                          p["edge_proj"]["w"], p["edge_proj"]["b"][None])

    nlayers = len(p["layers"])
    for i, ly in enumerate(p["layers"]):
        Gg = _gather_rows(G, idx_flat, 600)
        S = _msg_kernel(E, Gg, A, R, Rt,
                        ly["w_e"]["w"], ly["w_e"]["b"][None],
                        ly["w_m2"]["w"], ly["w_m2"]["b"][None])
        upd = (ly["w_m3"]["w"], ly["w_m3"]["b"][None],
               ly["ln1"]["g"][None], ly["ln1"]["b"][None],
               ly["ffn1"]["w"], ly["ffn1"]["b"][None],
               ly["ffn2"]["w"], ly["ffn2"]["b"][None],
               ly["ln2"]["g"][None], ly["ln2"]["b"][None])
        if i == nlayers - 1:
            (logits,) = _node_upd_kernel(
                V, S, upd,
                [(p["out_proj"]["w"], p["out_proj"]["b"][None])], [NAA])
            break
        nxt = p["layers"][i + 1]
        V, Ae, Ge, A, G = _node_upd_kernel(
            V, S, upd,
            [(ly["we_vi"]["w"], ly["we_vi"]["b"][None]),
             (ly["we_vj"]["w"], ly["we_vj"]["b"][None]),
             (nxt["w_vi"]["w"], nxt["w_vi"]["b"][None]),
             (nxt["w_vj"]["w"], nxt["w_vj"]["b"][None])],
            [DD, DD, DD, DD, DD])
        Gge = _gather_rows(Ge, idx_flat, 600)
        E = _edge_upd_kernel(E, Gge, Ae, R,
                             ly["we_e"]["w"], ly["we_e"]["b"][None],
                             ly["we_2"]["w"], ly["we_2"]["b"][None],
                             ly["ln_e"]["g"][None], ly["ln_e"]["b"][None])

    return logits[None, :, :]


# speedup vs baseline: 10.6483x; 1.0047x over previous
import functools

import numpy as np
import jax
import jax.numpy as jnp
from jax import lax
from jax.experimental import pallas as pl
from jax.experimental.pallas import tpu as pltpu
from jax.experimental.pallas import tpu_sc as plsc

NN = 10000
KK = 30
DD = 128
NRBF = 16
NAA = 20
NWL = 8
MINWL, MAXWL = 3.5, 12.0
MINRBF, MAXRBF = 2.0, 22.0

_RB = 200
_NPAN = 10
_PAN = 1024
_NCOL = _NPAN * _PAN

_NBE = 80
_EB = _NBE * KK
_EGRID = NN // _NBE

_NBV = 1000
_VGRID = NN // _NBV

_P = 307200
_NW = 32
_PERW = _P // _NW


def _gelu(x):
    return jax.nn.gelu(x)


def _mm(a, b):
    return lax.dot_general(a, b, (((1,), (0,)), ((), ())),
                           preferred_element_type=jnp.float32)


def _ln(x, g, b):
    mu = jnp.mean(x, axis=-1, keepdims=True)
    var = jnp.mean((x - mu) ** 2, axis=-1, keepdims=True)
    return (x - mu) * lax.rsqrt(var + 1e-5) * g + b


_NSUB = _PAN // 128
_TCA = 6


def _knn_body(car_ref, cat_ref, idx_ref, vals_ref, cv_ref, cc_ref):
    car = car_ref[...]
    x2r = ((car[:, 0:1] * car[:, 0:1] + car[:, 2:3] * car[:, 2:3])
           + car[:, 1:2] * car[:, 1:2])
    carb = car.astype(jnp.bfloat16).astype(jnp.float32)

    def init_vals():
        def init(p, _):
            catp = cat_ref[p]
            x2c = ((catp[0:1, :] * catp[0:1, :] + catp[2:3, :] * catp[2:3, :])
                   + catp[1:2, :] * catp[1:2, :])
            catb = catp.astype(jnp.bfloat16).astype(jnp.float32)
            p0 = carb[:, 0:1] * catb[0:1, :]
            p1 = carb[:, 1:2] * catb[1:2, :]
            p2 = carb[:, 2:3] * catb[2:3, :]
            s1 = p0 + p1
            bp = s1 - p0
            e1 = (p0 - (s1 - bp)) + (p1 - bp)
            s2 = s1 + p2
            bp2 = s2 - s1
            e2 = (s1 - (s2 - bp2)) + (p2 - bp2)
            dot = s2 + (e1 + e2)
            col = lax.broadcasted_iota(jnp.int32, (_RB, _PAN), 1) + p * _PAN
            d2 = x2r + x2c - 2.0 * dot
            vals_ref[p] = jnp.where(col >= NN, 1e30, d2)
            return _
        lax.fori_loop(0, _NPAN, init, 0)

    init_vals()

    laneio = lax.broadcasted_iota(jnp.int32, (_RB, 128), 1)
    srow = lax.broadcasted_iota(jnp.int32, (1, KK, _RB), 1)

    def cache_t(t, _):
        def scan_p(p, MC):
            M, Cc = MC
            for c in range(_NSUB):
                v = vals_ref[p, :, pl.ds(c * 128, 128)]
                colv = laneio + (p * _PAN + c * 128)
                better = v < M
                M = jnp.where(better, v, M)
                Cc = jnp.where(better, colv, Cc)
            return (M, Cc)
        M, Cc = lax.fori_loop(0, _NPAN, scan_p,
                              (jnp.full((_RB, 128), 1e30, jnp.float32),
                               jnp.full((_RB, 128), 2 ** 30, jnp.int32)))
        cv_ref[t] = M
        cc_ref[t] = Cc

        def mask_p(p, _c):
            for c in range(_NSUB):
                v = vals_ref[p, :, pl.ds(c * 128, 128)]
                colv = laneio + (p * _PAN + c * 128)
                vals_ref[p, :, pl.ds(c * 128, 128)] = jnp.where(
                    colv == Cc, 1e30, v)
            return _c
        lax.fori_loop(0, _NPAN, mask_p, 0)
        return _
    lax.fori_loop(0, _TCA, cache_t, 0)

    def extract(j, selcnt):
        mm = jnp.full((_RB, 1), 1e30, jnp.float32)
        for t in range(_TCA):
            mm = jnp.minimum(mm, jnp.min(cv_ref[t], axis=1, keepdims=True))
        sel = jnp.full((_RB,), 2 ** 30, jnp.int32)
        for t in range(_TCA):
            cand = jnp.where(cv_ref[t] == mm, cc_ref[t], jnp.int32(2 ** 30))
            sel = jnp.minimum(sel, jnp.min(cand, axis=1))
        idx_ref[...] = jnp.where(srow == j, sel[None, None, :], idx_ref[...])
        for t in range(_TCA):
            cv_ref[t] = jnp.where(cc_ref[t] == sel[:, None], 1e30, cv_ref[t])
        lane = lax.rem(sel, jnp.int32(128))
        return selcnt + jnp.where(laneio == lane[:, None], 1, 0)

    selcnt = lax.fori_loop(0, KK, extract,
                           jnp.zeros((_RB, 128), jnp.int32))

    @pl.when(jnp.any(selcnt >= _TCA))
    def _fallback():
        init_vals()

        def extract_full(j, _):
            def pmin(p, m):
                return jnp.minimum(
                    m, jnp.min(vals_ref[p], axis=1, keepdims=True))
            m = lax.fori_loop(0, _NPAN, pmin,
                              jnp.full((_RB, 1), 1e30, jnp.float32))

            def pargmin(p, best):
                v = vals_ref[p]
                col = lax.broadcasted_iota(jnp.int32, (_RB, _PAN), 1) + p * _PAN
                cand = jnp.where(v == m, col, jnp.int32(2 ** 30))
                return jnp.minimum(best, jnp.min(cand, axis=1))
            sel = lax.fori_loop(0, _NPAN, pargmin,
                                jnp.full((_RB,), 2 ** 30, jnp.int32))

            idx_ref[...] = jnp.where(srow == j, sel[None, None, :],
                                     idx_ref[...])

            def pupd(p, _c):
                v = vals_ref[p]
                col = lax.broadcasted_iota(jnp.int32, (_RB, _PAN), 1) + p * _PAN
                vals_ref[p] = jnp.where(col == sel[:, None], 1e30, v)
                return _c
            lax.fori_loop(0, _NPAN, pupd, 0)
            return _

        lax.fori_loop(0, KK, extract_full, 0)


def _knn(car, cat3):
    return pl.pallas_call(
        _knn_body,
        grid=(NN // _RB,),
        in_specs=[
            pl.BlockSpec((_RB, 8), lambda i: (i, 0)),
            pl.BlockSpec((_NPAN, 8, _PAN), lambda i: (0, 0, 0)),
        ],
        out_specs=pl.BlockSpec((1, KK, _RB), lambda i: (i, 0, 0)),
        out_shape=jax.ShapeDtypeStruct((NN // _RB, KK, _RB), jnp.int32),
        scratch_shapes=[pltpu.VMEM((_NPAN, _RB, _PAN), jnp.float32),
                        pltpu.VMEM((_TCA, _RB, 128), jnp.float32),
                        pltpu.VMEM((_TCA, _RB, 128), jnp.int32)],
    )(car, cat3)


@functools.lru_cache(maxsize=None)
def _sc_gather(dt, chunk, dtype):
    mesh = plsc.VectorSubcoreMesh(core_axis_name="c", subcore_axis_name="s")
    nit = _PERW // chunk

    @functools.partial(
        pl.kernel, mesh=mesh,
        out_type=jax.ShapeDtypeStruct((_P, dt), dtype),
        scratch_types=[
            pltpu.VMEM((_PERW,), jnp.int32),
            pltpu.VMEM((chunk, dt), dtype),
            pltpu.VMEM((chunk, dt), dtype),
            pltpu.SemaphoreType.DMA,
            pltpu.SemaphoreType.DMA,
        ],
    )
    def gk(table_hbm, idx_hbm, out_hbm, idx_v, rows0, rows1, sem0, sem1):
        wid = lax.axis_index("s") * 2 + lax.axis_index("c")
        base = wid * _PERW
        pltpu.sync_copy(idx_hbm.at[pl.ds(base, _PERW)], idx_v)
        rows = [rows0, rows1]
        sems = [sem0, sem1]
        prev = pltpu.async_copy(
            table_hbm.at[idx_v.at[pl.ds(0, chunk)]], rows[0], sems[0])
        for t in range(nit):
            nxt = None
            if t + 1 < nit:
                nxt = pltpu.async_copy(
                    table_hbm.at[idx_v.at[pl.ds((t + 1) * chunk, chunk)]],
                    rows[(t + 1) % 2], sems[(t + 1) % 2])
            prev.wait()
            pltpu.sync_copy(rows[t % 2],
                            out_hbm.at[pl.ds(base + t * chunk, chunk)])
            prev = nxt

    return gk


def _gather_rows(table, idx_pad, chunk):
    return _sc_gather(table.shape[1], chunk, table.dtype)(table, idx_pad)


def _msg_body(e_ref, gg_ref, a_ref, r_ref, rt_ref,
              we_ref, be_ref, w2_ref, b2_ref, out_ref):
    pre = (_mm(r_ref[...], a_ref[...]) + gg_ref[...].astype(jnp.float32)
           + _mm(e_ref[...], we_ref[...]) + be_ref[...])
    m = _gelu(pre)
    y = _gelu(_mm(m, w2_ref[...]) + b2_ref[...])
    out_ref[...] = _mm(rt_ref[...], y) * (1.0 / KK)


def _msg_kernel(E, Gg, A, R, Rt, we, be, w2, b2):
    return pl.pallas_call(
        _msg_body,
        grid=(_EGRID,),
        in_specs=[
            pl.BlockSpec((_EB, DD), lambda i: (i, 0)),
            pl.BlockSpec((_EB, DD), lambda i: (i, 0)),
            pl.BlockSpec((_NBE, DD), lambda i: (i, 0)),
            pl.BlockSpec((_EB, _NBE), lambda i: (0, 0)),
            pl.BlockSpec((_NBE, _EB), lambda i: (0, 0)),
            pl.BlockSpec((DD, DD), lambda i: (0, 0)),
            pl.BlockSpec((1, DD), lambda i: (0, 0)),
            pl.BlockSpec((DD, DD), lambda i: (0, 0)),
            pl.BlockSpec((1, DD), lambda i: (0, 0)),
        ],
        out_specs=pl.BlockSpec((_NBE, DD), lambda i: (i, 0)),
        out_shape=jax.ShapeDtypeStruct((NN, DD), jnp.float32),
    )(E, Gg, A, R, Rt, we, be, w2, b2)


def _edge_upd_body(e_ref, gg_ref, a_ref, r_ref,
                   we_ref, be_ref, w2_ref, b2_ref, g_ref, gb_ref, out_ref):
    pre = (_mm(r_ref[...], a_ref[...]) + gg_ref[...].astype(jnp.float32)
           + _mm(e_ref[...], we_ref[...]) + be_ref[...])
    h = _mm(_gelu(pre), w2_ref[...]) + b2_ref[...]
    out_ref[...] = _ln(e_ref[...] + h, g_ref[...], gb_ref[...])


def _edge_upd_kernel(E, Gg, A, R, we, be, w2, b2, g, gb):
    return pl.pallas_call(
        _edge_upd_body,
        grid=(_EGRID,),
        in_specs=[
            pl.BlockSpec((_EB, DD), lambda i: (i, 0)),
            pl.BlockSpec((_EB, DD), lambda i: (i, 0)),
            pl.BlockSpec((_NBE, DD), lambda i: (i, 0)),
            pl.BlockSpec((_EB, _NBE), lambda i: (0, 0)),
            pl.BlockSpec((DD, DD), lambda i: (0, 0)),
            pl.BlockSpec((1, DD), lambda i: (0, 0)),
            pl.BlockSpec((DD, DD), lambda i: (0, 0)),
            pl.BlockSpec((1, DD), lambda i: (0, 0)),
            pl.BlockSpec((1, DD), lambda i: (0, 0)),
            pl.BlockSpec((1, DD), lambda i: (0, 0)),
        ],
        out_specs=pl.BlockSpec((_EB, DD), lambda i: (i, 0)),
        out_shape=jax.ShapeDtypeStruct((NN * KK, DD), jnp.float32),
    )(E, Gg, A, R, we, be, w2, b2, g, gb)


def _node_upd_body(nouts, v_ref, s_ref, w3_ref, b3_ref, g1_ref, gb1_ref,
                   f1_ref, fb1_ref, f2_ref, fb2_ref, g2_ref, gb2_ref,
                   *rest):
    u = _ln(v_ref[...] + _mm(s_ref[...], w3_ref[...]) + b3_ref[...],
            g1_ref[...], gb1_ref[...])
    h = _mm(_gelu(_mm(u, f1_ref[...]) + fb1_ref[...]), f2_ref[...]) + fb2_ref[...]
    v2 = _ln(u + h, g2_ref[...], gb2_ref[...])
    wrefs = rest[:-nouts]
    orefs = rest[-nouts:]
    if nouts == 1:
        ow, ob = wrefs
        orefs[0][...] = _mm(v2, ow[...]) + ob[...]
    else:
        orefs[0][...] = v2
        for t in range(nouts - 1):
            w, b = wrefs[2 * t], wrefs[2 * t + 1]
            orefs[t + 1][...] = (_mm(v2, w[...]) + b[...]).astype(
                orefs[t + 1].dtype)


def _node_upd_kernel(V, S, upd_params, extra_ws, outs):
    nouts = len(outs)
    win = list(upd_params)
    for w, b in extra_ws:
        win += [w, b]
    wspecs = []
    for w in win:
        wspecs.append(pl.BlockSpec(w.shape, lambda i: (0, 0)))
    out_specs = [pl.BlockSpec((_NBV, d), lambda i: (i, 0)) for d, _ in outs]
    out_shape = [jax.ShapeDtypeStruct((NN, d), dt) for d, dt in outs]
    return pl.pallas_call(
        functools.partial(_node_upd_body, nouts),
        grid=(_VGRID,),
        in_specs=[
            pl.BlockSpec((_NBV, DD), lambda i: (i, 0)),
            pl.BlockSpec((_NBV, DD), lambda i: (i, 0)),
        ] + wspecs,
        out_specs=out_specs,
        out_shape=out_shape,
    )(V, S, *win)


def _feat_body(ang_ref, dv_ref, l_ref, wsin_ref, wcos_ref, wdv_ref, nb_ref,
               emb_ref, wvi_ref, bvi_ref, wvj_ref, bvj_ref,
               v_ref, a_ref, g_ref):
    ang = ang_ref[...]
    v0 = (_mm(jnp.sin(ang), wsin_ref[...]) + _mm(jnp.cos(ang), wcos_ref[...])
          + _mm(dv_ref[...], wdv_ref[...]) + nb_ref[...])
    lab = l_ref[...]
    onehot = (lab == lax.broadcasted_iota(jnp.int32, (_NBV, 21), 1)
              ).astype(jnp.float32)
    v0 = v0 + _mm(onehot, emb_ref[...])
    v_ref[...] = v0
    a_ref[...] = _mm(v0, wvi_ref[...]) + bvi_ref[...]
    g_ref[...] = (_mm(v0, wvj_ref[...]) + bvj_ref[...]).astype(g_ref.dtype)


def _feat_kernel(ang, dv, lab, wsin, wcos, wdv, nb, emb, wvi, bvi, wvj, bvj):
    ws = [wsin, wcos, wdv, nb, emb, wvi, bvi, wvj, bvj]
    wspecs = [pl.BlockSpec(w.shape, lambda i: (0, 0)) for w in ws]
    return pl.pallas_call(
        _feat_body,
        grid=(_VGRID,),
        in_specs=[
            pl.BlockSpec((_NBV, 24), lambda i: (i, 0)),
            pl.BlockSpec((_NBV, 8), lambda i: (i, 0)),
            pl.BlockSpec((_NBV, 1), lambda i: (i, 0)),
        ] + wspecs,
        out_specs=[pl.BlockSpec((_NBV, DD), lambda i: (i, 0))] * 3,
        out_shape=[jax.ShapeDtypeStruct((NN, DD), jnp.float32)] * 3,
    )(ang, dv, lab, *ws)


def _edge_feat_body(caj_ref, ca_ref, r_ref, cen_ref, ew_ref, eb_ref, out_ref):
    cai = _mm(r_ref[...], ca_ref[...])
    diff = cai - caj_ref[...]
    d2 = jnp.sum(diff * diff, axis=1, keepdims=True)
    d = jnp.sqrt(d2 + 1e-8)
    sigma = (MAXRBF - MINRBF) / NRBF
    z = (d - cen_ref[...]) / sigma
    rbf = jnp.exp(-(z * z))
    out_ref[...] = _mm(rbf, ew_ref[...]) + eb_ref[...]


def _edge_feat_kernel(Caj, Ca16, R, cen, ew, eb):
    return pl.pallas_call(
        _edge_feat_body,
        grid=(_EGRID,),
        in_specs=[
            pl.BlockSpec((_EB, DD), lambda i: (i, 0)),
            pl.BlockSpec((_NBE, DD), lambda i: (i, 0)),
            pl.BlockSpec((_EB, _NBE), lambda i: (0, 0)),
            pl.BlockSpec((1, 16), lambda i: (0, 0)),
            pl.BlockSpec((16, DD), lambda i: (0, 0)),
            pl.BlockSpec((1, DD), lambda i: (0, 0)),
        ],
        out_specs=pl.BlockSpec((_EB, DD), lambda i: (i, 0)),
        out_shape=jax.ShapeDtypeStruct((NN * KK, DD), jnp.float32),
    )(Caj, Ca16, R, cen, ew, eb)


_R_NP = np.kron(np.eye(_NBE, dtype=np.float32), np.ones((KK, 1), np.float32))


def kernel(C, L, chain_idxs, params):
    del chain_idxs
    C0 = C[0]
    Nat, Ca, Cc = C0[:, 0, :], C0[:, 1, :], C0[:, 2, :]
    bv = Ca - Nat
    cv = Cc - Ca
    av = jnp.cross(bv, cv)
    Cb = -0.58273431 * av + 0.56802827 * bv - 0.54067466 * cv + Ca
    dvec = Cb - Ca
    dvec = dvec / (jnp.linalg.norm(dvec, axis=-1, keepdims=True) + 1e-8)
    dvec8 = jnp.pad(dvec, ((0, 0), (0, 5)))

    wlv = jnp.geomspace(MINWL, MAXWL, NWL)
    ang = (Ca[:, :, None] / wlv).reshape(NN, 3 * NWL)

    car = jnp.pad(Ca, ((0, 0), (0, 5)))
    cat3 = jnp.pad(Ca.T, ((0, 5), (0, _NCOL - NN))).reshape(8, _NPAN, _PAN)
    cat3 = jnp.transpose(cat3, (1, 0, 2))
    idx3 = _knn(car, cat3)
    idx = jnp.transpose(idx3, (0, 2, 1)).reshape(NN, KK)
    idx_flat = jnp.concatenate(
        [idx.reshape(-1), jnp.arange(_P - NN * KK, dtype=jnp.int32) % NN])

    p = params
    R = jnp.asarray(_R_NP)
    Rt = R.T
    lab2 = L[0][:, None].astype(jnp.int32)

    nw = p["node_proj"]["w"]
    rows_sin = np.array([d * 16 + w for d in range(3) for w in range(NWL)])
    rows_cos = rows_sin + NWL
    wsin = nw[rows_sin]
    wcos = nw[rows_cos]
    wdv = jnp.pad(nw[48:51], ((0, 5), (0, 0)))
    nb = p["node_proj"]["b"][None, :]
    l0 = p["layers"][0]
    V, A, G = _feat_kernel(
        ang, dvec8, lab2, wsin, wcos, wdv, nb, p["label_embed"],
        l0["w_vi"]["w"], l0["w_vi"]["b"][None], l0["w_vj"]["w"],
        l0["w_vj"]["b"][None])

    Ca16 = jnp.pad(Ca, ((0, 0), (0, DD - 3)))
    Caj = _gather_rows(Ca16, idx_flat, 400)
    cen = jnp.linspace(MINRBF, MAXRBF, NRBF)[None, :]
    E = _edge_feat_kernel(Caj, Ca16, R, cen,
                          p["edge_proj"]["w"], p["edge_proj"]["b"][None])

    nlayers = len(p["layers"])
    for i, ly in enumerate(p["layers"]):
        Gg = _gather_rows(G, idx_flat, 400)
        S = _msg_kernel(E, Gg, A, R, Rt,
                        ly["w_e"]["w"], ly["w_e"]["b"][None],
                        ly["w_m2"]["w"], ly["w_m2"]["b"][None])
        upd = (ly["w_m3"]["w"], ly["w_m3"]["b"][None],
               ly["ln1"]["g"][None], ly["ln1"]["b"][None],
               ly["ffn1"]["w"], ly["ffn1"]["b"][None],
               ly["ffn2"]["w"], ly["ffn2"]["b"][None],
               ly["ln2"]["g"][None], ly["ln2"]["b"][None])
        if i == nlayers - 1:
            (logits,) = _node_upd_kernel(
                V, S, upd,
                [(p["out_proj"]["w"], p["out_proj"]["b"][None])],
                [(NAA, jnp.float32)])
            break
        nxt = p["layers"][i + 1]
        V, Ae, Ge, A, G = _node_upd_kernel(
            V, S, upd,
            [(ly["we_vi"]["w"], ly["we_vi"]["b"][None]),
             (ly["we_vj"]["w"], ly["we_vj"]["b"][None]),
             (nxt["w_vi"]["w"], nxt["w_vi"]["b"][None]),
             (nxt["w_vj"]["w"], nxt["w_vj"]["b"][None])],
            [(DD, jnp.float32)] * 5)
        Gge = _gather_rows(Ge, idx_flat, 400)
        E = _edge_upd_kernel(E, Gge, Ae, R,
                             ly["we_e"]["w"], ly["we_e"]["b"][None],
                             ly["we_2"]["w"], ly["we_2"]["b"][None],
                             ly["ln_e"]["g"][None], ly["ln_e"]["b"][None])

    return logits[None, :, :]


# speedup vs baseline: 10.9463x; 1.0280x over previous
import functools

import numpy as np
import jax
import jax.numpy as jnp
from jax import lax
from jax.experimental import pallas as pl
from jax.experimental.pallas import tpu as pltpu
from jax.experimental.pallas import tpu_sc as plsc

NN = 10000
KK = 30
DD = 128
NRBF = 16
NAA = 20
NWL = 8
MINWL, MAXWL = 3.5, 12.0
MINRBF, MAXRBF = 2.0, 22.0

_RB = 200
_NPAN = 10
_PAN = 1024
_NCOL = _NPAN * _PAN

_NBE = 80
_EB = _NBE * KK
_EGRID = NN // _NBE

_NBV = 1000
_VGRID = NN // _NBV

_P = 307200
_NW = 32
_PERW = _P // _NW


def _gelu(x):
    return jax.nn.gelu(x)


def _mm(a, b):
    return lax.dot_general(a, b, (((1,), (0,)), ((), ())),
                           preferred_element_type=jnp.float32)


def _ln(x, g, b):
    mu = jnp.mean(x, axis=-1, keepdims=True)
    var = jnp.mean((x - mu) ** 2, axis=-1, keepdims=True)
    return (x - mu) * lax.rsqrt(var + 1e-5) * g + b


_NSUB = _PAN // 128
_TCA = 6


def _knn_body(car_ref, cat_ref, idx_ref, vals_ref, cv_ref, cc_ref):
    car = car_ref[...]
    x2r = ((car[:, 0:1] * car[:, 0:1] + car[:, 2:3] * car[:, 2:3])
           + car[:, 1:2] * car[:, 1:2])
    carb = car.astype(jnp.bfloat16).astype(jnp.float32)

    def init_vals():
        def init(p, _):
            catp = cat_ref[p]
            x2c = ((catp[0:1, :] * catp[0:1, :] + catp[2:3, :] * catp[2:3, :])
                   + catp[1:2, :] * catp[1:2, :])
            catb = catp.astype(jnp.bfloat16).astype(jnp.float32)
            p0 = carb[:, 0:1] * catb[0:1, :]
            p1 = carb[:, 1:2] * catb[1:2, :]
            p2 = carb[:, 2:3] * catb[2:3, :]
            s1 = p0 + p1
            bp = s1 - p0
            e1 = (p0 - (s1 - bp)) + (p1 - bp)
            s2 = s1 + p2
            bp2 = s2 - s1
            e2 = (s1 - (s2 - bp2)) + (p2 - bp2)
            dot = s2 + (e1 + e2)
            col = lax.broadcasted_iota(jnp.int32, (_RB, _PAN), 1) + p * _PAN
            d2 = x2r + x2c - 2.0 * dot
            vals_ref[p] = jnp.where(col >= NN, 1e30, d2)
            return _
        lax.fori_loop(0, _NPAN, init, 0)

    init_vals()

    laneio = lax.broadcasted_iota(jnp.int32, (_RB, 128), 1)
    srow = lax.broadcasted_iota(jnp.int32, (1, KK, _RB), 1)

    def cache_t(t, _):
        def scan_p(p, MC):
            M, Cc = MC
            for c in range(_NSUB):
                v = vals_ref[p, :, pl.ds(c * 128, 128)]
                colv = laneio + (p * _PAN + c * 128)
                better = v < M
                M = jnp.where(better, v, M)
                Cc = jnp.where(better, colv, Cc)
            return (M, Cc)
        M, Cc = lax.fori_loop(0, _NPAN, scan_p,
                              (jnp.full((_RB, 128), 1e30, jnp.float32),
                               jnp.full((_RB, 128), 2 ** 30, jnp.int32)))
        cv_ref[t] = M
        cc_ref[t] = Cc

        def mask_p(p, _c):
            for c in range(_NSUB):
                v = vals_ref[p, :, pl.ds(c * 128, 128)]
                colv = laneio + (p * _PAN + c * 128)
                vals_ref[p, :, pl.ds(c * 128, 128)] = jnp.where(
                    colv == Cc, 1e30, v)
            return _c
        lax.fori_loop(0, _NPAN, mask_p, 0)
        return _
    lax.fori_loop(0, _TCA, cache_t, 0)

    def extract(j, selcnt):
        mm = jnp.full((_RB, 1), 1e30, jnp.float32)
        for t in range(_TCA):
            mm = jnp.minimum(mm, jnp.min(cv_ref[t], axis=1, keepdims=True))
        sel = jnp.full((_RB,), 2 ** 30, jnp.int32)
        for t in range(_TCA):
            cand = jnp.where(cv_ref[t] == mm, cc_ref[t], jnp.int32(2 ** 30))
            sel = jnp.minimum(sel, jnp.min(cand, axis=1))
        idx_ref[...] = jnp.where(srow == j, sel[None, None, :], idx_ref[...])
        for t in range(_TCA):
            cv_ref[t] = jnp.where(cc_ref[t] == sel[:, None], 1e30, cv_ref[t])
        lane = lax.rem(sel, jnp.int32(128))
        return selcnt + jnp.where(laneio == lane[:, None], 1, 0)

    selcnt = lax.fori_loop(0, KK, extract,
                           jnp.zeros((_RB, 128), jnp.int32))

    @pl.when(jnp.any(selcnt >= _TCA))
    def _fallback():
        init_vals()

        def extract_full(j, _):
            def pmin(p, m):
                return jnp.minimum(
                    m, jnp.min(vals_ref[p], axis=1, keepdims=True))
            m = lax.fori_loop(0, _NPAN, pmin,
                              jnp.full((_RB, 1), 1e30, jnp.float32))

            def pargmin(p, best):
                v = vals_ref[p]
                col = lax.broadcasted_iota(jnp.int32, (_RB, _PAN), 1) + p * _PAN
                cand = jnp.where(v == m, col, jnp.int32(2 ** 30))
                return jnp.minimum(best, jnp.min(cand, axis=1))
            sel = lax.fori_loop(0, _NPAN, pargmin,
                                jnp.full((_RB,), 2 ** 30, jnp.int32))

            idx_ref[...] = jnp.where(srow == j, sel[None, None, :],
                                     idx_ref[...])

            def pupd(p, _c):
                v = vals_ref[p]
                col = lax.broadcasted_iota(jnp.int32, (_RB, _PAN), 1) + p * _PAN
                vals_ref[p] = jnp.where(col == sel[:, None], 1e30, v)
                return _c
            lax.fori_loop(0, _NPAN, pupd, 0)
            return _

        lax.fori_loop(0, KK, extract_full, 0)


def _knn(car, cat3):
    return pl.pallas_call(
        _knn_body,
        grid=(NN // _RB,),
        in_specs=[
            pl.BlockSpec((_RB, 8), lambda i: (i, 0)),
            pl.BlockSpec((_NPAN, 8, _PAN), lambda i: (0, 0, 0)),
        ],
        out_specs=pl.BlockSpec((1, KK, _RB), lambda i: (i, 0, 0)),
        out_shape=jax.ShapeDtypeStruct((NN // _RB, KK, _RB), jnp.int32),
        scratch_shapes=[pltpu.VMEM((_NPAN, _RB, _PAN), jnp.float32),
                        pltpu.VMEM((_TCA, _RB, 128), jnp.float32),
                        pltpu.VMEM((_TCA, _RB, 128), jnp.int32)],
    )(car, cat3)


@functools.lru_cache(maxsize=None)
def _sc_gather(dt, chunk, dtype):
    mesh = plsc.VectorSubcoreMesh(core_axis_name="c", subcore_axis_name="s")
    nit = _PERW // chunk

    @functools.partial(
        pl.kernel, mesh=mesh,
        out_type=jax.ShapeDtypeStruct((_P, dt), dtype),
        scratch_types=[
            pltpu.VMEM((_PERW,), jnp.int32),
            pltpu.VMEM((chunk, dt), dtype),
            pltpu.VMEM((chunk, dt), dtype),
            pltpu.SemaphoreType.DMA,
            pltpu.SemaphoreType.DMA,
        ],
    )
    def gk(table_hbm, idx_hbm, out_hbm, idx_v, rows0, rows1, sem0, sem1):
        wid = lax.axis_index("s") * 2 + lax.axis_index("c")
        base = wid * _PERW
        pltpu.sync_copy(idx_hbm.at[pl.ds(base, _PERW)], idx_v)
        rows = [rows0, rows1]
        sems = [sem0, sem1]
        prev = pltpu.async_copy(
            table_hbm.at[idx_v.at[pl.ds(0, chunk)]], rows[0], sems[0])
        for t in range(nit):
            nxt = None
            if t + 1 < nit:
                nxt = pltpu.async_copy(
                    table_hbm.at[idx_v.at[pl.ds((t + 1) * chunk, chunk)]],
                    rows[(t + 1) % 2], sems[(t + 1) % 2])
            prev.wait()
            pltpu.sync_copy(rows[t % 2],
                            out_hbm.at[pl.ds(base + t * chunk, chunk)])
            prev = nxt

    return gk


def _gather_rows(table, idx_pad, chunk):
    return _sc_gather(table.shape[1], chunk, table.dtype)(table, idx_pad)


def _msg_body(e_ref, gg_ref, a_ref, r_ref, rt_ref,
              we_ref, be_ref, w2_ref, b2_ref, out_ref):
    pre = (_mm(r_ref[...], a_ref[...]) + gg_ref[...].astype(jnp.float32)
           + _mm(e_ref[...].astype(jnp.float32), we_ref[...]) + be_ref[...])
    m = _gelu(pre)
    y = _gelu(_mm(m, w2_ref[...]) + b2_ref[...])
    out_ref[...] = _mm(rt_ref[...], y) * (1.0 / KK)


def _msg_kernel(E, Gg, A, R, Rt, we, be, w2, b2):
    return pl.pallas_call(
        _msg_body,
        grid=(_EGRID,),
        in_specs=[
            pl.BlockSpec((_EB, DD), lambda i: (i, 0)),
            pl.BlockSpec((_EB, DD), lambda i: (i, 0)),
            pl.BlockSpec((_NBE, DD), lambda i: (i, 0)),
            pl.BlockSpec((_EB, _NBE), lambda i: (0, 0)),
            pl.BlockSpec((_NBE, _EB), lambda i: (0, 0)),
            pl.BlockSpec((DD, DD), lambda i: (0, 0)),
            pl.BlockSpec((1, DD), lambda i: (0, 0)),
            pl.BlockSpec((DD, DD), lambda i: (0, 0)),
            pl.BlockSpec((1, DD), lambda i: (0, 0)),
        ],
        out_specs=pl.BlockSpec((_NBE, DD), lambda i: (i, 0)),
        out_shape=jax.ShapeDtypeStruct((NN, DD), jnp.float32),
    )(E, Gg, A, R, Rt, we, be, w2, b2)


def _edge_upd_body(e_ref, gg_ref, a_ref, r_ref,
                   we_ref, be_ref, w2_ref, b2_ref, g_ref, gb_ref, out_ref):
    e32 = e_ref[...].astype(jnp.float32)
    pre = (_mm(r_ref[...], a_ref[...]) + gg_ref[...].astype(jnp.float32)
           + _mm(e32, we_ref[...]) + be_ref[...])
    h = _mm(_gelu(pre), w2_ref[...]) + b2_ref[...]
    out_ref[...] = _ln(e32 + h, g_ref[...], gb_ref[...]).astype(jnp.bfloat16)


def _edge_upd_kernel(E, Gg, A, R, we, be, w2, b2, g, gb):
    return pl.pallas_call(
        _edge_upd_body,
        grid=(_EGRID,),
        in_specs=[
            pl.BlockSpec((_EB, DD), lambda i: (i, 0)),
            pl.BlockSpec((_EB, DD), lambda i: (i, 0)),
            pl.BlockSpec((_NBE, DD), lambda i: (i, 0)),
            pl.BlockSpec((_EB, _NBE), lambda i: (0, 0)),
            pl.BlockSpec((DD, DD), lambda i: (0, 0)),
            pl.BlockSpec((1, DD), lambda i: (0, 0)),
            pl.BlockSpec((DD, DD), lambda i: (0, 0)),
            pl.BlockSpec((1, DD), lambda i: (0, 0)),
            pl.BlockSpec((1, DD), lambda i: (0, 0)),
            pl.BlockSpec((1, DD), lambda i: (0, 0)),
        ],
        out_specs=pl.BlockSpec((_EB, DD), lambda i: (i, 0)),
        out_shape=jax.ShapeDtypeStruct((NN * KK, DD), jnp.bfloat16),
    )(E, Gg, A, R, we, be, w2, b2, g, gb)


def _node_upd_body(nouts, v_ref, s_ref, w3_ref, b3_ref, g1_ref, gb1_ref,
                   f1_ref, fb1_ref, f2_ref, fb2_ref, g2_ref, gb2_ref,
                   *rest):
    u = _ln(v_ref[...] + _mm(s_ref[...], w3_ref[...]) + b3_ref[...],
            g1_ref[...], gb1_ref[...])
    h = _mm(_gelu(_mm(u, f1_ref[...]) + fb1_ref[...]), f2_ref[...]) + fb2_ref[...]
    v2 = _ln(u + h, g2_ref[...], gb2_ref[...])
    wrefs = rest[:-nouts]
    orefs = rest[-nouts:]
    if nouts == 1:
        ow, ob = wrefs
        orefs[0][...] = _mm(v2, ow[...]) + ob[...]
    else:
        orefs[0][...] = v2
        for t in range(nouts - 1):
            w, b = wrefs[2 * t], wrefs[2 * t + 1]
            orefs[t + 1][...] = (_mm(v2, w[...]) + b[...]).astype(
                orefs[t + 1].dtype)


def _node_upd_kernel(V, S, upd_params, extra_ws, outs):
    nouts = len(outs)
    win = list(upd_params)
    for w, b in extra_ws:
        win += [w, b]
    wspecs = []
    for w in win:
        wspecs.append(pl.BlockSpec(w.shape, lambda i: (0, 0)))
    out_specs = [pl.BlockSpec((_NBV, d), lambda i: (i, 0)) for d, _ in outs]
    out_shape = [jax.ShapeDtypeStruct((NN, d), dt) for d, dt in outs]
    return pl.pallas_call(
        functools.partial(_node_upd_body, nouts),
        grid=(_VGRID,),
        in_specs=[
            pl.BlockSpec((_NBV, DD), lambda i: (i, 0)),
            pl.BlockSpec((_NBV, DD), lambda i: (i, 0)),
        ] + wspecs,
        out_specs=out_specs,
        out_shape=out_shape,
    )(V, S, *win)


def _feat_body(ang_ref, dv_ref, l_ref, wsin_ref, wcos_ref, wdv_ref, nb_ref,
               emb_ref, wvi_ref, bvi_ref, wvj_ref, bvj_ref,
               v_ref, a_ref, g_ref):
    ang = ang_ref[...]
    v0 = (_mm(jnp.sin(ang), wsin_ref[...]) + _mm(jnp.cos(ang), wcos_ref[...])
          + _mm(dv_ref[...], wdv_ref[...]) + nb_ref[...])
    lab = l_ref[...]
    onehot = (lab == lax.broadcasted_iota(jnp.int32, (_NBV, 21), 1)
              ).astype(jnp.float32)
    v0 = v0 + _mm(onehot, emb_ref[...])
    v_ref[...] = v0
    a_ref[...] = _mm(v0, wvi_ref[...]) + bvi_ref[...]
    g_ref[...] = (_mm(v0, wvj_ref[...]) + bvj_ref[...]).astype(g_ref.dtype)


def _feat_kernel(ang, dv, lab, wsin, wcos, wdv, nb, emb, wvi, bvi, wvj, bvj):
    ws = [wsin, wcos, wdv, nb, emb, wvi, bvi, wvj, bvj]
    wspecs = [pl.BlockSpec(w.shape, lambda i: (0, 0)) for w in ws]
    return pl.pallas_call(
        _feat_body,
        grid=(_VGRID,),
        in_specs=[
            pl.BlockSpec((_NBV, 24), lambda i: (i, 0)),
            pl.BlockSpec((_NBV, 8), lambda i: (i, 0)),
            pl.BlockSpec((_NBV, 1), lambda i: (i, 0)),
        ] + wspecs,
        out_specs=[pl.BlockSpec((_NBV, DD), lambda i: (i, 0))] * 3,
        out_shape=[jax.ShapeDtypeStruct((NN, DD), jnp.float32)] * 3,
    )(ang, dv, lab, *ws)


def _edge_feat_body(caj_ref, ca_ref, r_ref, cen_ref, ew_ref, eb_ref, out_ref):
    cai = _mm(r_ref[...], ca_ref[...])
    diff = cai - caj_ref[...]
    d2 = jnp.sum(diff * diff, axis=1, keepdims=True)
    d = jnp.sqrt(d2 + 1e-8)
    sigma = (MAXRBF - MINRBF) / NRBF
    z = (d - cen_ref[...]) / sigma
    rbf = jnp.exp(-(z * z))
    out_ref[...] = (_mm(rbf, ew_ref[...]) + eb_ref[...]).astype(jnp.bfloat16)


def _edge_feat_kernel(Caj, Ca16, R, cen, ew, eb):
    return pl.pallas_call(
        _edge_feat_body,
        grid=(_EGRID,),
        in_specs=[
            pl.BlockSpec((_EB, DD), lambda i: (i, 0)),
            pl.BlockSpec((_NBE, DD), lambda i: (i, 0)),
            pl.BlockSpec((_EB, _NBE), lambda i: (0, 0)),
            pl.BlockSpec((1, 16), lambda i: (0, 0)),
            pl.BlockSpec((16, DD), lambda i: (0, 0)),
            pl.BlockSpec((1, DD), lambda i: (0, 0)),
        ],
        out_specs=pl.BlockSpec((_EB, DD), lambda i: (i, 0)),
        out_shape=jax.ShapeDtypeStruct((NN * KK, DD), jnp.bfloat16),
    )(Caj, Ca16, R, cen, ew, eb)


_R_NP = np.kron(np.eye(_NBE, dtype=np.float32), np.ones((KK, 1), np.float32))


def kernel(C, L, chain_idxs, params):
    del chain_idxs
    C0 = C[0]
    Nat, Ca, Cc = C0[:, 0, :], C0[:, 1, :], C0[:, 2, :]
    bv = Ca - Nat
    cv = Cc - Ca
    av = jnp.cross(bv, cv)
    Cb = -0.58273431 * av + 0.56802827 * bv - 0.54067466 * cv + Ca
    dvec = Cb - Ca
    dvec = dvec / (jnp.linalg.norm(dvec, axis=-1, keepdims=True) + 1e-8)
    dvec8 = jnp.pad(dvec, ((0, 0), (0, 5)))

    wlv = jnp.geomspace(MINWL, MAXWL, NWL)
    ang = (Ca[:, :, None] / wlv).reshape(NN, 3 * NWL)

    car = jnp.pad(Ca, ((0, 0), (0, 5)))
    cat3 = jnp.pad(Ca.T, ((0, 5), (0, _NCOL - NN))).reshape(8, _NPAN, _PAN)
    cat3 = jnp.transpose(cat3, (1, 0, 2))
    idx3 = _knn(car, cat3)
    idx = jnp.transpose(idx3, (0, 2, 1)).reshape(NN, KK)
    idx_flat = jnp.concatenate(
        [idx.reshape(-1), jnp.arange(_P - NN * KK, dtype=jnp.int32) % NN])

    p = params
    R = jnp.asarray(_R_NP)
    Rt = R.T
    lab2 = L[0][:, None].astype(jnp.int32)

    nw = p["node_proj"]["w"]
    rows_sin = np.array([d * 16 + w for d in range(3) for w in range(NWL)])
    rows_cos = rows_sin + NWL
    wsin = nw[rows_sin]
    wcos = nw[rows_cos]
    wdv = jnp.pad(nw[48:51], ((0, 5), (0, 0)))
    nb = p["node_proj"]["b"][None, :]
    l0 = p["layers"][0]
    V, A, G = _feat_kernel(
        ang, dvec8, lab2, wsin, wcos, wdv, nb, p["label_embed"],
        l0["w_vi"]["w"], l0["w_vi"]["b"][None], l0["w_vj"]["w"],
        l0["w_vj"]["b"][None])

    Ca16 = jnp.pad(Ca, ((0, 0), (0, DD - 3)))
    Caj = _gather_rows(Ca16, idx_flat, 400)
    cen = jnp.linspace(MINRBF, MAXRBF, NRBF)[None, :]
    E = _edge_feat_kernel(Caj, Ca16, R, cen,
                          p["edge_proj"]["w"], p["edge_proj"]["b"][None])

    nlayers = len(p["layers"])
    for i, ly in enumerate(p["layers"]):
        Gg = _gather_rows(G, idx_flat, 400)
        S = _msg_kernel(E, Gg, A, R, Rt,
                        ly["w_e"]["w"], ly["w_e"]["b"][None],
                        ly["w_m2"]["w"], ly["w_m2"]["b"][None])
        upd = (ly["w_m3"]["w"], ly["w_m3"]["b"][None],
               ly["ln1"]["g"][None], ly["ln1"]["b"][None],
               ly["ffn1"]["w"], ly["ffn1"]["b"][None],
               ly["ffn2"]["w"], ly["ffn2"]["b"][None],
               ly["ln2"]["g"][None], ly["ln2"]["b"][None])
        if i == nlayers - 1:
            (logits,) = _node_upd_kernel(
                V, S, upd,
                [(p["out_proj"]["w"], p["out_proj"]["b"][None])],
                [(NAA, jnp.float32)])
            break
        nxt = p["layers"][i + 1]
        V, Ae, Ge, A, G = _node_upd_kernel(
            V, S, upd,
            [(ly["we_vi"]["w"], ly["we_vi"]["b"][None]),
             (ly["we_vj"]["w"], ly["we_vj"]["b"][None]),
             (nxt["w_vi"]["w"], nxt["w_vi"]["b"][None]),
             (nxt["w_vj"]["w"], nxt["w_vj"]["b"][None])],
            [(DD, jnp.float32)] * 5)
        Gge = _gather_rows(Ge, idx_flat, 400)
        E = _edge_upd_kernel(E, Gge, Ae, R,
                             ly["we_e"]["w"], ly["we_e"]["b"][None],
                             ly["we_2"]["w"], ly["we_2"]["b"][None],
                             ly["ln_e"]["g"][None], ly["ln_e"]["b"][None])

    return logits[None, :, :]


# speedup vs baseline: 11.0366x; 1.0082x over previous
import functools

import numpy as np
import jax
import jax.numpy as jnp
from jax import lax
from jax.experimental import pallas as pl
from jax.experimental.pallas import tpu as pltpu
from jax.experimental.pallas import tpu_sc as plsc

NN = 10000
KK = 30
DD = 128
NRBF = 16
NAA = 20
NWL = 8
MINWL, MAXWL = 3.5, 12.0
MINRBF, MAXRBF = 2.0, 22.0

_RB = 200
_NPAN = 10
_PAN = 1024
_NCOL = _NPAN * _PAN

_NBE = 80
_EB = _NBE * KK
_EGRID = NN // _NBE

_NBV = 1000
_VGRID = NN // _NBV

_P = 307200
_NW = 32
_PERW = _P // _NW


def _gelu(x):
    return jax.nn.gelu(x)


def _mm(a, b):
    return lax.dot_general(a, b, (((1,), (0,)), ((), ())),
                           preferred_element_type=jnp.float32)


def _ln(x, g, b):
    mu = jnp.mean(x, axis=-1, keepdims=True)
    var = jnp.mean((x - mu) ** 2, axis=-1, keepdims=True)
    return (x - mu) * lax.rsqrt(var + 1e-5) * g + b


_NSUB = _PAN // 128
_TCA = 6


def _knn_body(car_ref, cat_ref, idx_ref, vals_ref, cv_ref, cc_ref):
    car = car_ref[...]
    x2r = ((car[:, 0:1] * car[:, 0:1] + car[:, 2:3] * car[:, 2:3])
           + car[:, 1:2] * car[:, 1:2])
    carb = car.astype(jnp.bfloat16).astype(jnp.float32)

    def init_vals():
        def init(p, _):
            catp = cat_ref[p]
            x2c = ((catp[0:1, :] * catp[0:1, :] + catp[2:3, :] * catp[2:3, :])
                   + catp[1:2, :] * catp[1:2, :])
            catb = catp.astype(jnp.bfloat16).astype(jnp.float32)
            p0 = carb[:, 0:1] * catb[0:1, :]
            p1 = carb[:, 1:2] * catb[1:2, :]
            p2 = carb[:, 2:3] * catb[2:3, :]
            s1 = p0 + p1
            bp = s1 - p0
            e1 = (p0 - (s1 - bp)) + (p1 - bp)
            s2 = s1 + p2
            bp2 = s2 - s1
            e2 = (s1 - (s2 - bp2)) + (p2 - bp2)
            dot = s2 + (e1 + e2)
            col = lax.broadcasted_iota(jnp.int32, (_RB, _PAN), 1) + p * _PAN
            d2 = x2r + x2c - 2.0 * dot
            vals_ref[p] = jnp.where(col >= NN, 1e30, d2)
            return _
        lax.fori_loop(0, _NPAN, init, 0)

    init_vals()

    laneio = lax.broadcasted_iota(jnp.int32, (_RB, 128), 1)
    srow = lax.broadcasted_iota(jnp.int32, (1, KK, _RB), 1)

    def scan_p(p, carry):
        Ms, Cs = carry
        for c in range(_NSUB):
            v = vals_ref[p, :, pl.ds(c * 128, 128)]
            cv = laneio + (p * _PAN + c * 128)
            lt = [v < M for M in Ms]
            nM, nC = [jnp.where(lt[0], v, Ms[0])], [jnp.where(lt[0], cv, Cs[0])]
            for i in range(1, _TCA):
                nM.append(jnp.where(lt[i - 1], Ms[i - 1],
                                    jnp.where(lt[i], v, Ms[i])))
                nC.append(jnp.where(lt[i - 1], Cs[i - 1],
                                    jnp.where(lt[i], cv, Cs[i])))
            Ms, Cs = tuple(nM), tuple(nC)
        return (Ms, Cs)

    Ms, Cs = lax.fori_loop(
        0, _NPAN, scan_p,
        (tuple(jnp.full((_RB, 128), 1e30, jnp.float32)
               for _ in range(_TCA)),
         tuple(jnp.full((_RB, 128), 2 ** 30, jnp.int32)
               for _ in range(_TCA))))
    for t in range(_TCA):
        cv_ref[t] = Ms[t]
        cc_ref[t] = Cs[t]

    def extract(j, selcnt):
        mm = jnp.full((_RB, 1), 1e30, jnp.float32)
        for t in range(_TCA):
            mm = jnp.minimum(mm, jnp.min(cv_ref[t], axis=1, keepdims=True))
        sel = jnp.full((_RB,), 2 ** 30, jnp.int32)
        for t in range(_TCA):
            cand = jnp.where(cv_ref[t] == mm, cc_ref[t], jnp.int32(2 ** 30))
            sel = jnp.minimum(sel, jnp.min(cand, axis=1))
        idx_ref[...] = jnp.where(srow == j, sel[None, None, :], idx_ref[...])
        for t in range(_TCA):
            cv_ref[t] = jnp.where(cc_ref[t] == sel[:, None], 1e30, cv_ref[t])
        lane = lax.rem(sel, jnp.int32(128))
        return selcnt + jnp.where(laneio == lane[:, None], 1, 0)

    selcnt = lax.fori_loop(0, KK, extract,
                           jnp.zeros((_RB, 128), jnp.int32))

    @pl.when(jnp.any(selcnt >= _TCA))
    def _fallback():
        def extract_full(j, _):
            def pmin(p, m):
                return jnp.minimum(
                    m, jnp.min(vals_ref[p], axis=1, keepdims=True))
            m = lax.fori_loop(0, _NPAN, pmin,
                              jnp.full((_RB, 1), 1e30, jnp.float32))

            def pargmin(p, best):
                v = vals_ref[p]
                col = lax.broadcasted_iota(jnp.int32, (_RB, _PAN), 1) + p * _PAN
                cand = jnp.where(v == m, col, jnp.int32(2 ** 30))
                return jnp.minimum(best, jnp.min(cand, axis=1))
            sel = lax.fori_loop(0, _NPAN, pargmin,
                                jnp.full((_RB,), 2 ** 30, jnp.int32))

            idx_ref[...] = jnp.where(srow == j, sel[None, None, :],
                                     idx_ref[...])

            def pupd(p, _c):
                v = vals_ref[p]
                col = lax.broadcasted_iota(jnp.int32, (_RB, _PAN), 1) + p * _PAN
                vals_ref[p] = jnp.where(col == sel[:, None], 1e30, v)
                return _c
            lax.fori_loop(0, _NPAN, pupd, 0)
            return _

        lax.fori_loop(0, KK, extract_full, 0)


def _knn(car, cat3):
    return pl.pallas_call(
        _knn_body,
        grid=(NN // _RB,),
        in_specs=[
            pl.BlockSpec((_RB, 8), lambda i: (i, 0)),
            pl.BlockSpec((_NPAN, 8, _PAN), lambda i: (0, 0, 0)),
        ],
        out_specs=pl.BlockSpec((1, KK, _RB), lambda i: (i, 0, 0)),
        out_shape=jax.ShapeDtypeStruct((NN // _RB, KK, _RB), jnp.int32),
        scratch_shapes=[pltpu.VMEM((_NPAN, _RB, _PAN), jnp.float32),
                        pltpu.VMEM((_TCA, _RB, 128), jnp.float32),
                        pltpu.VMEM((_TCA, _RB, 128), jnp.int32)],
    )(car, cat3)


@functools.lru_cache(maxsize=None)
def _sc_gather(dt, chunk, dtype):
    mesh = plsc.VectorSubcoreMesh(core_axis_name="c", subcore_axis_name="s")
    nit = _PERW // chunk

    @functools.partial(
        pl.kernel, mesh=mesh,
        out_type=jax.ShapeDtypeStruct((_P, dt), dtype),
        scratch_types=[
            pltpu.VMEM((_PERW,), jnp.int32),
            pltpu.VMEM((chunk, dt), dtype),
            pltpu.VMEM((chunk, dt), dtype),
            pltpu.SemaphoreType.DMA,
            pltpu.SemaphoreType.DMA,
        ],
    )
    def gk(table_hbm, idx_hbm, out_hbm, idx_v, rows0, rows1, sem0, sem1):
        wid = lax.axis_index("s") * 2 + lax.axis_index("c")
        base = wid * _PERW
        pltpu.sync_copy(idx_hbm.at[pl.ds(base, _PERW)], idx_v)
        rows = [rows0, rows1]
        sems = [sem0, sem1]
        prev = pltpu.async_copy(
            table_hbm.at[idx_v.at[pl.ds(0, chunk)]], rows[0], sems[0])
        for t in range(nit):
            nxt = None
            if t + 1 < nit:
                nxt = pltpu.async_copy(
                    table_hbm.at[idx_v.at[pl.ds((t + 1) * chunk, chunk)]],
                    rows[(t + 1) % 2], sems[(t + 1) % 2])
            prev.wait()
            pltpu.sync_copy(rows[t % 2],
                            out_hbm.at[pl.ds(base + t * chunk, chunk)])
            prev = nxt

    return gk


def _gather_rows(table, idx_pad, chunk):
    return _sc_gather(table.shape[1], chunk, table.dtype)(table, idx_pad)


def _msg_body(e_ref, gg_ref, a_ref, r_ref, rt_ref,
              we_ref, be_ref, w2_ref, b2_ref, out_ref):
    pre = (_mm(r_ref[...], a_ref[...]) + gg_ref[...].astype(jnp.float32)
           + _mm(e_ref[...].astype(jnp.float32), we_ref[...]) + be_ref[...])
    m = _gelu(pre)
    y = _gelu(_mm(m, w2_ref[...]) + b2_ref[...])
    out_ref[...] = _mm(rt_ref[...], y) * (1.0 / KK)


def _msg_kernel(E, Gg, A, R, Rt, we, be, w2, b2):
    return pl.pallas_call(
        _msg_body,
        grid=(_EGRID,),
        in_specs=[
            pl.BlockSpec((_EB, DD), lambda i: (i, 0)),
            pl.BlockSpec((_EB, DD), lambda i: (i, 0)),
            pl.BlockSpec((_NBE, DD), lambda i: (i, 0)),
            pl.BlockSpec((_EB, _NBE), lambda i: (0, 0)),
            pl.BlockSpec((_NBE, _EB), lambda i: (0, 0)),
            pl.BlockSpec((DD, DD), lambda i: (0, 0)),
            pl.BlockSpec((1, DD), lambda i: (0, 0)),
            pl.BlockSpec((DD, DD), lambda i: (0, 0)),
            pl.BlockSpec((1, DD), lambda i: (0, 0)),
        ],
        out_specs=pl.BlockSpec((_NBE, DD), lambda i: (i, 0)),
        out_shape=jax.ShapeDtypeStruct((NN, DD), jnp.float32),
    )(E, Gg, A, R, Rt, we, be, w2, b2)


def _edge_upd_body(e_ref, gg_ref, a_ref, r_ref,
                   we_ref, be_ref, w2_ref, b2_ref, g_ref, gb_ref, out_ref):
    e32 = e_ref[...].astype(jnp.float32)
    pre = (_mm(r_ref[...], a_ref[...]) + gg_ref[...].astype(jnp.float32)
           + _mm(e32, we_ref[...]) + be_ref[...])
    h = _mm(_gelu(pre), w2_ref[...]) + b2_ref[...]
    out_ref[...] = _ln(e32 + h, g_ref[...], gb_ref[...]).astype(jnp.bfloat16)


def _edge_upd_kernel(E, Gg, A, R, we, be, w2, b2, g, gb):
    return pl.pallas_call(
        _edge_upd_body,
        grid=(_EGRID,),
        in_specs=[
            pl.BlockSpec((_EB, DD), lambda i: (i, 0)),
            pl.BlockSpec((_EB, DD), lambda i: (i, 0)),
            pl.BlockSpec((_NBE, DD), lambda i: (i, 0)),
            pl.BlockSpec((_EB, _NBE), lambda i: (0, 0)),
            pl.BlockSpec((DD, DD), lambda i: (0, 0)),
            pl.BlockSpec((1, DD), lambda i: (0, 0)),
            pl.BlockSpec((DD, DD), lambda i: (0, 0)),
            pl.BlockSpec((1, DD), lambda i: (0, 0)),
            pl.BlockSpec((1, DD), lambda i: (0, 0)),
            pl.BlockSpec((1, DD), lambda i: (0, 0)),
        ],
        out_specs=pl.BlockSpec((_EB, DD), lambda i: (i, 0)),
        out_shape=jax.ShapeDtypeStruct((NN * KK, DD), jnp.bfloat16),
    )(E, Gg, A, R, we, be, w2, b2, g, gb)


def _node_upd_body(nouts, v_ref, s_ref, w3_ref, b3_ref, g1_ref, gb1_ref,
                   f1_ref, fb1_ref, f2_ref, fb2_ref, g2_ref, gb2_ref,
                   *rest):
    u = _ln(v_ref[...] + _mm(s_ref[...], w3_ref[...]) + b3_ref[...],
            g1_ref[...], gb1_ref[...])
    h = _mm(_gelu(_mm(u, f1_ref[...]) + fb1_ref[...]), f2_ref[...]) + fb2_ref[...]
    v2 = _ln(u + h, g2_ref[...], gb2_ref[...])
    wrefs = rest[:-nouts]
    orefs = rest[-nouts:]
    if nouts == 1:
        ow, ob = wrefs
        orefs[0][...] = _mm(v2, ow[...]) + ob[...]
    else:
        orefs[0][...] = v2
        for t in range(nouts - 1):
            w, b = wrefs[2 * t], wrefs[2 * t + 1]
            orefs[t + 1][...] = (_mm(v2, w[...]) + b[...]).astype(
                orefs[t + 1].dtype)


def _node_upd_kernel(V, S, upd_params, extra_ws, outs):
    nouts = len(outs)
    win = list(upd_params)
    for w, b in extra_ws:
        win += [w, b]
    wspecs = []
    for w in win:
        wspecs.append(pl.BlockSpec(w.shape, lambda i: (0, 0)))
    out_specs = [pl.BlockSpec((_NBV, d), lambda i: (i, 0)) for d, _ in outs]
    out_shape = [jax.ShapeDtypeStruct((NN, d), dt) for d, dt in outs]
    return pl.pallas_call(
        functools.partial(_node_upd_body, nouts),
        grid=(_VGRID,),
        in_specs=[
            pl.BlockSpec((_NBV, DD), lambda i: (i, 0)),
            pl.BlockSpec((_NBV, DD), lambda i: (i, 0)),
        ] + wspecs,
        out_specs=out_specs,
        out_shape=out_shape,
    )(V, S, *win)


def _feat_body(ang_ref, dv_ref, l_ref, wsin_ref, wcos_ref, wdv_ref, nb_ref,
               emb_ref, wvi_ref, bvi_ref, wvj_ref, bvj_ref,
               v_ref, a_ref, g_ref):
    ang = ang_ref[...]
    v0 = (_mm(jnp.sin(ang), wsin_ref[...]) + _mm(jnp.cos(ang), wcos_ref[...])
          + _mm(dv_ref[...], wdv_ref[...]) + nb_ref[...])
    lab = l_ref[...]
    onehot = (lab == lax.broadcasted_iota(jnp.int32, (_NBV, 21), 1)
              ).astype(jnp.float32)
    v0 = v0 + _mm(onehot, emb_ref[...])
    v_ref[...] = v0
    a_ref[...] = _mm(v0, wvi_ref[...]) + bvi_ref[...]
    g_ref[...] = (_mm(v0, wvj_ref[...]) + bvj_ref[...]).astype(g_ref.dtype)


def _feat_kernel(ang, dv, lab, wsin, wcos, wdv, nb, emb, wvi, bvi, wvj, bvj):
    ws = [wsin, wcos, wdv, nb, emb, wvi, bvi, wvj, bvj]
    wspecs = [pl.BlockSpec(w.shape, lambda i: (0, 0)) for w in ws]
    return pl.pallas_call(
        _feat_body,
        grid=(_VGRID,),
        in_specs=[
            pl.BlockSpec((_NBV, 24), lambda i: (i, 0)),
            pl.BlockSpec((_NBV, 8), lambda i: (i, 0)),
            pl.BlockSpec((_NBV, 1), lambda i: (i, 0)),
        ] + wspecs,
        out_specs=[pl.BlockSpec((_NBV, DD), lambda i: (i, 0))] * 3,
        out_shape=[jax.ShapeDtypeStruct((NN, DD), jnp.float32)] * 3,
    )(ang, dv, lab, *ws)


def _edge_feat_body(caj_ref, ca_ref, r_ref, cen_ref, ew_ref, eb_ref, out_ref):
    cai = _mm(r_ref[...], ca_ref[...])
    diff = cai - caj_ref[...]
    d2 = jnp.sum(diff * diff, axis=1, keepdims=True)
    d = jnp.sqrt(d2 + 1e-8)
    sigma = (MAXRBF - MINRBF) / NRBF
    z = (d - cen_ref[...]) / sigma
    rbf = jnp.exp(-(z * z))
    out_ref[...] = (_mm(rbf, ew_ref[...]) + eb_ref[...]).astype(jnp.bfloat16)


def _edge_feat_kernel(Caj, Ca16, R, cen, ew, eb):
    return pl.pallas_call(
        _edge_feat_body,
        grid=(_EGRID,),
        in_specs=[
            pl.BlockSpec((_EB, DD), lambda i: (i, 0)),
            pl.BlockSpec((_NBE, DD), lambda i: (i, 0)),
            pl.BlockSpec((_EB, _NBE), lambda i: (0, 0)),
            pl.BlockSpec((1, 16), lambda i: (0, 0)),
            pl.BlockSpec((16, DD), lambda i: (0, 0)),
            pl.BlockSpec((1, DD), lambda i: (0, 0)),
        ],
        out_specs=pl.BlockSpec((_EB, DD), lambda i: (i, 0)),
        out_shape=jax.ShapeDtypeStruct((NN * KK, DD), jnp.bfloat16),
    )(Caj, Ca16, R, cen, ew, eb)


_R_NP = np.kron(np.eye(_NBE, dtype=np.float32), np.ones((KK, 1), np.float32))


def kernel(C, L, chain_idxs, params):
    del chain_idxs
    C0 = C[0]
    Nat, Ca, Cc = C0[:, 0, :], C0[:, 1, :], C0[:, 2, :]
    bv = Ca - Nat
    cv = Cc - Ca
    av = jnp.cross(bv, cv)
    Cb = -0.58273431 * av + 0.56802827 * bv - 0.54067466 * cv + Ca
    dvec = Cb - Ca
    dvec = dvec / (jnp.linalg.norm(dvec, axis=-1, keepdims=True) + 1e-8)
    dvec8 = jnp.pad(dvec, ((0, 0), (0, 5)))

    wlv = jnp.geomspace(MINWL, MAXWL, NWL)
    ang = (Ca[:, :, None] / wlv).reshape(NN, 3 * NWL)

    car = jnp.pad(Ca, ((0, 0), (0, 5)))
    cat3 = jnp.pad(Ca.T, ((0, 5), (0, _NCOL - NN))).reshape(8, _NPAN, _PAN)
    cat3 = jnp.transpose(cat3, (1, 0, 2))
    idx3 = _knn(car, cat3)
    idx = jnp.transpose(idx3, (0, 2, 1)).reshape(NN, KK)
    idx_flat = jnp.concatenate(
        [idx.reshape(-1), jnp.arange(_P - NN * KK, dtype=jnp.int32) % NN])

    p = params
    R = jnp.asarray(_R_NP)
    Rt = R.T
    lab2 = L[0][:, None].astype(jnp.int32)

    nw = p["node_proj"]["w"]
    rows_sin = np.array([d * 16 + w for d in range(3) for w in range(NWL)])
    rows_cos = rows_sin + NWL
    wsin = nw[rows_sin]
    wcos = nw[rows_cos]
    wdv = jnp.pad(nw[48:51], ((0, 5), (0, 0)))
    nb = p["node_proj"]["b"][None, :]
    l0 = p["layers"][0]
    V, A, G = _feat_kernel(
        ang, dvec8, lab2, wsin, wcos, wdv, nb, p["label_embed"],
        l0["w_vi"]["w"], l0["w_vi"]["b"][None], l0["w_vj"]["w"],
        l0["w_vj"]["b"][None])

    Ca16 = jnp.pad(Ca, ((0, 0), (0, DD - 3)))
    Caj = _gather_rows(Ca16, idx_flat, 400)
    cen = jnp.linspace(MINRBF, MAXRBF, NRBF)[None, :]
    E = _edge_feat_kernel(Caj, Ca16, R, cen,
                          p["edge_proj"]["w"], p["edge_proj"]["b"][None])

    nlayers = len(p["layers"])
    for i, ly in enumerate(p["layers"]):
        Gg = _gather_rows(G, idx_flat, 400)
        S = _msg_kernel(E, Gg, A, R, Rt,
                        ly["w_e"]["w"], ly["w_e"]["b"][None],
                        ly["w_m2"]["w"], ly["w_m2"]["b"][None])
        upd = (ly["w_m3"]["w"], ly["w_m3"]["b"][None],
               ly["ln1"]["g"][None], ly["ln1"]["b"][None],
               ly["ffn1"]["w"], ly["ffn1"]["b"][None],
               ly["ffn2"]["w"], ly["ffn2"]["b"][None],
               ly["ln2"]["g"][None], ly["ln2"]["b"][None])
        if i == nlayers - 1:
            (logits,) = _node_upd_kernel(
                V, S, upd,
                [(p["out_proj"]["w"], p["out_proj"]["b"][None])],
                [(NAA, jnp.float32)])
            break
        nxt = p["layers"][i + 1]
        V, Ae, Ge, A, G = _node_upd_kernel(
            V, S, upd,
            [(ly["we_vi"]["w"], ly["we_vi"]["b"][None]),
             (ly["we_vj"]["w"], ly["we_vj"]["b"][None]),
             (nxt["w_vi"]["w"], nxt["w_vi"]["b"][None]),
             (nxt["w_vj"]["w"], nxt["w_vj"]["b"][None])],
            [(DD, jnp.float32)] * 5)
        Gge = _gather_rows(Ge, idx_flat, 400)
        E = _edge_upd_kernel(E, Gge, Ae, R,
                             ly["we_e"]["w"], ly["we_e"]["b"][None],
                             ly["we_2"]["w"], ly["we_2"]["b"][None],
                             ly["ln_e"]["g"][None], ly["ln_e"]["b"][None])

    return logits[None, :, :]
